# Initial kernel scaffold; baseline (speedup 1.0000x reference)
import jax
import jax.numpy as jnp
from jax.experimental import pallas as pl


def kernel(x, edge_index, A, weight, temperature, norm_energies, W, b):
    raise NotImplementedError("write your pallas kernel here")


# speedup vs baseline: 3.0449x; 3.0449x over previous
import functools
import jax
import jax.numpy as jnp
from jax import lax
from jax.experimental import pallas as pl
from jax.experimental.pallas import tpu as pltpu
from jax.experimental.pallas import tpu_sc as plsc

N = 10000
E = 160000
D = 256
DH = 128
EPS = 1e-12

NC = 2
NS = 16
NW = NC * NS

EPT_PAD = 5120
E_PAD = EPT_PAD * NW
BP1 = 64
NBLK1 = EPT_PAD // BP1
BP2 = 64
NBLK2 = EPT_PAD // BP2
N_PAD = 10240
ROWS_PER_TILE = N_PAD // NS

_mesh = plsc.VectorSubcoreMesh(
    core_axis_name="c", subcore_axis_name="s", num_cores=NC, num_subcores=NS)


def _matmul_body(x_ref, w_ref, o_ref):
  o_ref[0] = jnp.dot(x_ref[...], w_ref[...],
                     preferred_element_type=jnp.float32)


def _matmul(x, W):
  RT = 400
  return pl.pallas_call(
      _matmul_body,
      grid=(N // RT, 2),
      in_specs=[
          pl.BlockSpec((RT, D), lambda i, c: (i, 0)),
          pl.BlockSpec((D, DH), lambda i, c: (0, c)),
      ],
      out_specs=pl.BlockSpec((1, RT, DH), lambda i, c: (c, i, 0)),
      out_shape=jax.ShapeDtypeStruct((2, N, DH), jnp.float32),
  )(x, W)


def _p1_body(x_hbm, srcp, dstp, ap, flagp, zeros_np,
             en_out, deg_out,
             sidx_v, didx_v, a_v, f_v, xs_v, xd_v, e_buf, tbuf,
             en_acc, deg_acc, sem):
  c = lax.axis_index("c")
  s = lax.axis_index("s")
  w = c * NS + s

  @pl.when(s == 0)
  def _init():
    pltpu.sync_copy(zeros_np, en_acc)
    pltpu.sync_copy(zeros_np, deg_acc)

  plsc.subcore_barrier()

  def block_body(b, carry):
    pltpu.sync_copy(srcp.at[w, b], sidx_v)
    pltpu.sync_copy(dstp.at[w, b], didx_v)
    pltpu.sync_copy(ap.at[w, b], a_v)
    pltpu.sync_copy(flagp.at[w, b], f_v)
    d1 = pltpu.async_copy(x_hbm.at[sidx_v], xs_v, sem)
    d2 = pltpu.async_copy(x_hbm.at[didx_v], xd_v, sem)
    d1.wait()
    d2.wait()

    lane17 = lax.iota(jnp.int32, 16) * 17

    def group_body(g, carry2):
      a16 = a_v[pl.ds(g * 16, 16)]
      for e16 in range(16):
        e = g * 16 + e16
        acc = jnp.zeros((16,), jnp.float32)
        for k in range(D // 16):
          dxy = xs_v[e, pl.ds(k * 16, 16)] - xd_v[e, pl.ds(k * 16, 16)]
          acc = acc + dxy * dxy
        tbuf[pl.ds(e16 * 17, 16)] = acc
      esum = jnp.zeros((16,), jnp.float32)
      for ccol in range(16):
        esum = esum + plsc.load_gather(tbuf, [lane17 + ccol])
      e_buf[pl.ds(g * 16, 16)] = a16 * esum
      return carry2

    lax.fori_loop(0, BP1 // 16, group_body, 0)
    pltpu.sync_copy(e_buf, en_acc.at[didx_v], add=True)
    pltpu.sync_copy(f_v, deg_acc.at[didx_v], add=True)
    return carry

  lax.fori_loop(0, NBLK1, block_body, 0)

  plsc.subcore_barrier()

  @pl.when(s == 0)
  def _writeout():
    pltpu.sync_copy(en_acc, en_out.at[c])
    pltpu.sync_copy(deg_acc, deg_out.at[c])


def _p1(x, srcp, dstp, ap, flagp, zeros_np):
  f32 = jnp.float32
  return pl.kernel(
      _p1_body,
      out_type=[
          jax.ShapeDtypeStruct((NC, N_PAD), f32),
          jax.ShapeDtypeStruct((NC, N_PAD), f32),
      ],
      mesh=_mesh,
      compiler_params=pltpu.CompilerParams(needs_layout_passes=False),
      scratch_types=[
          pltpu.VMEM((BP1,), jnp.int32),
          pltpu.VMEM((BP1,), jnp.int32),
          pltpu.VMEM((BP1,), f32),
          pltpu.VMEM((BP1,), f32),
          pltpu.VMEM((BP1, D), f32),
          pltpu.VMEM((BP1, D), f32),
          pltpu.VMEM((BP1,), f32),
          pltpu.VMEM((16 * 17,), f32),
          pltpu.VMEM_SHARED((N_PAD,), f32),
          pltpu.VMEM_SHARED((N_PAD,), f32),
          pltpu.SemaphoreType.DMA,
      ],
  )(x, srcp, dstp, ap, flagp, zeros_np)


def _node_body(ep_ref, dp_ref, t_ref, w_ref, ne_ref, dinv_ref, cn_ref,
               dsq_ref):
  en = ep_ref[0] + ep_ref[1]
  deg = dp_ref[0] + dp_ref[1] + 1.0
  row = lax.broadcasted_iota(jnp.int32, (N_PAD // 128, 128), 0)
  col = lax.broadcasted_iota(jnp.int32, (N_PAD // 128, 128), 1)
  valid = (row * 128 + col) < N

  T = t_ref[0, 0]
  wgt = w_ref[0, 0]
  do_norm = ne_ref[0, 0] != 0
  neg_inf = jnp.float32(-jnp.inf)

  m = jnp.max(jnp.where(valid, en, neg_inf))
  r = 1.0 / (m + EPS)
  en_n = jnp.where(do_norm, en * r, en)

  z = -en_n / T
  zmax = jnp.max(jnp.where(valid, z, neg_inf))
  ez = jnp.where(valid, jnp.exp(z - zmax), 0.0)
  p = ez / jnp.sum(ez)
  g = -(jnp.log(p + EPS) + p / (p + EPS))
  pg = jnp.sum(jnp.where(valid, p * g, 0.0))
  u = (-1.0 / T) * p * (g - pg)
  sum_ue = jnp.sum(jnp.where(valid, u * en, 0.0))
  is_max = jnp.where(valid & (en == m), 1.0, 0.0)
  ties = jnp.sum(is_max)
  v = jnp.where(do_norm, u * r - (r * r) * sum_ue * is_max / ties, u)

  dinv = lax.rsqrt(deg)
  dinv_ref[...] = dinv
  cn_ref[...] = 2.0 * wgt * v
  dsq_ref[...] = 1.0 / deg


def _node_math(en_part, deg_part, temperature, weight, norm_energies):
  f32 = jnp.float32
  shp = (N_PAD // 128, 128)
  return pl.pallas_call(
      _node_body,
      in_specs=[
          pl.BlockSpec((NC,) + shp, lambda: (0, 0, 0)),
          pl.BlockSpec((NC,) + shp, lambda: (0, 0, 0)),
          pl.BlockSpec((1, 1), lambda: (0, 0)),
          pl.BlockSpec((1, 1), lambda: (0, 0)),
          pl.BlockSpec((1, 1), lambda: (0, 0)),
      ],
      out_specs=[
          pl.BlockSpec(shp, lambda: (0, 0)),
          pl.BlockSpec(shp, lambda: (0, 0)),
          pl.BlockSpec(shp, lambda: (0, 0)),
      ],
      out_shape=[
          jax.ShapeDtypeStruct(shp, f32),
          jax.ShapeDtypeStruct(shp, f32),
          jax.ShapeDtypeStruct(shp, f32),
      ],
  )(en_part.reshape((NC,) + shp), deg_part.reshape((NC,) + shp),
    temperature.reshape(1, 1), weight.reshape(1, 1),
    jnp.asarray(norm_energies, jnp.int32).reshape(1, 1))


def _p2_body(xcat, hcat, srcp, dstp, ap, flagp, dinv_hbm, cn_hbm,
             zeros_nd, zeros_np,
             acc_out, s_out,
             sidx_v, didx_v, sidx2_v, didx2_v, a_v, f_v,
             alpha_v, beta_v, sval_v,
             hs_v, xs_v, xd_v,
             dinv_t, cn_t,
             accum, sacc, sem):
  c = lax.axis_index("c")
  s = lax.axis_index("s")

  pltpu.sync_copy(dinv_hbm, dinv_t)
  pltpu.sync_copy(cn_hbm, cn_t)
  pltpu.sync_copy(zeros_nd.at[pl.ds(s * ROWS_PER_TILE, ROWS_PER_TILE)],
                  accum.at[pl.ds(s * ROWS_PER_TILE, ROWS_PER_TILE)])

  @pl.when((c == 0) & (s == 0))
  def _init_s():
    pltpu.sync_copy(zeros_np, sacc)

  plsc.subcore_barrier()

  row_off = c * N

  def block_body(wj, b):
    pltpu.sync_copy(srcp.at[wj, b], sidx_v)
    pltpu.sync_copy(dstp.at[wj, b], didx_v)
    pltpu.sync_copy(ap.at[wj, b], a_v)
    pltpu.sync_copy(flagp.at[wj, b], f_v)

    for k in range(BP2 // 16):
      sl = pl.ds(k * 16, 16)
      sidx2_v[sl] = sidx_v[sl] + row_off
      didx2_v[sl] = didx_v[sl] + row_off

    d1 = pltpu.async_copy(hcat.at[sidx2_v], hs_v, sem)
    d2 = pltpu.async_copy(xcat.at[sidx2_v], xs_v, sem)
    d3 = pltpu.async_copy(xcat.at[didx2_v], xd_v, sem)
    d1.wait()
    d2.wait()
    d3.wait()

    for k in range(BP2 // 16):
      sl = pl.ds(k * 16, 16)
      idx_s = sidx_v[sl]
      idx_d = didx_v[sl]
      dv_s = plsc.load_gather(dinv_t, [idx_s])
      dv_d = plsc.load_gather(dinv_t, [idx_d])
      cn_d = plsc.load_gather(cn_t, [idx_d])
      sv = a_v[sl] * cn_d
      alpha_v[sl] = f_v[sl] * dv_s * dv_d
      beta_v[sl] = -sv
      sval_v[sl] = sv

    @pl.when(c == 0)
    def _s_accum():
      pltpu.sync_copy(sval_v, sacc.at[sidx_v], add=True)
      pltpu.sync_copy(sval_v, sacc.at[didx_v], add=True)

    def group_body(g, carry2):
      a16 = alpha_v[pl.ds(g * 16, 16)]
      b16 = beta_v[pl.ds(g * 16, 16)]
      for e16 in range(16):
        e = g * 16 + e16
        ae = a16[e16]
        be = b16[e16]
        for k in range(DH // 16):
          sl = pl.ds(k * 16, 16)
          xs_v[e, sl] = ae * hs_v[e, sl] + be * xs_v[e, sl]
          xd_v[e, sl] = be * xd_v[e, sl]
      return carry2

    lax.fori_loop(0, BP2 // 16, group_body, 0)

    pltpu.sync_copy(xs_v, accum.at[didx_v], add=True)
    pltpu.sync_copy(xd_v, accum.at[sidx_v], add=True)

  def outer(b, carry):
    block_body(2 * s, b)
    block_body(2 * s + 1, b)
    return carry

  lax.fori_loop(0, NBLK2, outer, 0)

  plsc.subcore_barrier()

  rsl = pl.ds(s * ROWS_PER_TILE, ROWS_PER_TILE)
  pltpu.sync_copy(accum.at[rsl], acc_out.at[c, rsl])

  @pl.when((c == 0) & (s == 0))
  def _write_s():
    pltpu.sync_copy(sacc, s_out)


def _p2(xcat, hcat, srcp, dstp, ap, flagp, dinv_n, cn_n, zeros_nd, zeros_np):
  f32 = jnp.float32
  i32 = jnp.int32
  return pl.kernel(
      _p2_body,
      out_type=[
          jax.ShapeDtypeStruct((NC, N_PAD, DH), f32),
          jax.ShapeDtypeStruct((N_PAD,), f32),
      ],
      mesh=_mesh,
      compiler_params=pltpu.CompilerParams(needs_layout_passes=False),
      scratch_types=[
          pltpu.VMEM((BP2,), i32),
          pltpu.VMEM((BP2,), i32),
          pltpu.VMEM((BP2,), i32),
          pltpu.VMEM((BP2,), i32),
          pltpu.VMEM((BP2,), f32),
          pltpu.VMEM((BP2,), f32),
          pltpu.VMEM((BP2,), f32),
          pltpu.VMEM((BP2,), f32),
          pltpu.VMEM((BP2,), f32),
          pltpu.VMEM((BP2, DH), f32),
          pltpu.VMEM((BP2, DH), f32),
          pltpu.VMEM((BP2, DH), f32),
          pltpu.VMEM((N,), f32),
          pltpu.VMEM((N,), f32),
          pltpu.VMEM_SHARED((N_PAD, DH), f32),
          pltpu.VMEM_SHARED((N_PAD,), f32),
          pltpu.SemaphoreType.DMA,
      ],
  )(xcat, hcat, srcp, dstp, ap, flagp, dinv_n, cn_n, zeros_nd, zeros_np)


def _combine_body(acc_ref, h_ref, x_ref, dsq_ref, s_ref, b_ref, o_ref):
  dsq = dsq_ref[...]
  sc = s_ref[...]
  bias = b_ref[...]
  lo = acc_ref[0] + dsq * h_ref[0] + sc * x_ref[:, :DH] + bias[:, :DH]
  hi = acc_ref[1] + dsq * h_ref[1] + sc * x_ref[:, DH:] + bias[:, DH:]
  o_ref[:, :DH] = lo
  o_ref[:, DH:] = hi


def _combine(acc, h3, x, dsq_col, s_col, b2d):
  RT = 400
  f32 = jnp.float32
  return pl.pallas_call(
      _combine_body,
      grid=(N // RT,),
      in_specs=[
          pl.BlockSpec((NC, RT, DH), lambda i: (0, i, 0)),
          pl.BlockSpec((NC, RT, DH), lambda i: (0, i, 0)),
          pl.BlockSpec((RT, D), lambda i: (i, 0)),
          pl.BlockSpec((RT, 1), lambda i: (i, 0)),
          pl.BlockSpec((RT, 1), lambda i: (i, 0)),
          pl.BlockSpec((1, D), lambda i: (0, 0)),
      ],
      out_specs=pl.BlockSpec((RT, D), lambda i: (i, 0)),
      out_shape=jax.ShapeDtypeStruct((N, D), f32),
  )(acc, h3, x, dsq_col, s_col, b2d)


@jax.jit
def _run(x, edge_index, A, weight, temperature, norm_energies, W, b):
  f32 = jnp.float32
  i32 = jnp.int32
  src = edge_index[0]
  dst = edge_index[1]

  pad = E_PAD - E
  shp_e = (NW, NBLK1, BP1)
  srcp = jnp.concatenate([src, jnp.zeros((pad,), i32)]).reshape(shp_e)
  dstp = jnp.concatenate([dst, jnp.zeros((pad,), i32)]).reshape(shp_e)
  ap = jnp.concatenate([A, jnp.zeros((pad,), f32)]).reshape(shp_e)
  flagp = jnp.concatenate([jnp.ones((E,), f32), jnp.zeros((pad,), f32)]
                          ).reshape(shp_e)
  zeros_np = jnp.zeros((N_PAD,), f32)
  zeros_nd = jnp.zeros((N_PAD, DH), f32)

  h3 = _matmul(x, W)
  hcat = h3.reshape(2 * N, DH)
  xcat = x.reshape(N, 2, DH).transpose(1, 0, 2).reshape(2 * N, DH)

  en_part, deg_part = _p1(x, srcp, dstp, ap, flagp, zeros_np)
  dinv2d, cn2d, dsq2d = _node_math(en_part, deg_part, temperature, weight,
                                   norm_energies)
  dinv_n = dinv2d.reshape(-1)[:N]
  cn_n = cn2d.reshape(-1)[:N]

  acc, s_n = _p2(xcat, hcat, srcp, dstp, ap, flagp, dinv_n, cn_n,
                 zeros_nd, zeros_np)

  out = _combine(acc, h3, x, dsq2d.reshape(-1)[:N].reshape(N, 1),
                 s_n[:N].reshape(N, 1), b.reshape(1, D))
  return out


def kernel(x, edge_index, A, weight, temperature, norm_energies, W, b):
  return _run(x, edge_index, A, weight, temperature, norm_energies, W, b)


# speedup vs baseline: 3.1723x; 1.0418x over previous
import jax
import jax.numpy as jnp
from jax import lax
from jax.experimental import pallas as pl
from jax.experimental.pallas import tpu as pltpu
from jax.experimental.pallas import tpu_sc as plsc

N = 10000
E = 160000
D = 256
DH = 128
EPS = 1e-12

NC = 2
NS = 16
NW = NC * NS

BLOCK = 128
EPT_PAD = 5120
NBLK = EPT_PAD // BLOCK
E_PAD = EPT_PAD * NW
N_PAD = 10240
N_ACC = 10112
ROWS_ACC = N_ACC // NS

_mesh = plsc.VectorSubcoreMesh(
    core_axis_name="c", subcore_axis_name="s", num_cores=NC, num_subcores=NS)
_sc_params = pltpu.CompilerParams(needs_layout_passes=False)


def _matmul_body(x_ref, w_ref, o_ref):
  o_ref[0] = jnp.dot(x_ref[...], w_ref[...],
                     preferred_element_type=jnp.float32)


def _matmul(x, W):
  RT = 400
  return pl.pallas_call(
      _matmul_body,
      grid=(N // RT, 2),
      in_specs=[
          pl.BlockSpec((RT, D), lambda i, c: (i, 0)),
          pl.BlockSpec((D, DH), lambda i, c: (0, c)),
      ],
      out_specs=pl.BlockSpec((1, RT, DH), lambda i, c: (c, i, 0)),
      out_shape=jax.ShapeDtypeStruct((2, N, DH), jnp.float32),
  )(x, W)


def _p1_body(x_hbm, srcp, dstp, ap, flagp, zeros_np,
             en_out, deg_out,
             sidx_v, didx_v, a_v, f_v, xs_v, xd_v, e_buf, tbuf,
             en_acc, deg_acc, sem):
  c = lax.axis_index("c")
  s = lax.axis_index("s")
  w = c * NS + s

  @pl.when(s == 0)
  def _init():
    pltpu.sync_copy(zeros_np, en_acc)
    pltpu.sync_copy(zeros_np, deg_acc)

  plsc.subcore_barrier()

  def block_body(b, carry):
    pltpu.sync_copy(srcp.at[w, b], sidx_v)
    pltpu.sync_copy(dstp.at[w, b], didx_v)
    pltpu.sync_copy(ap.at[w, b], a_v)
    pltpu.sync_copy(flagp.at[w, b], f_v)
    d1 = pltpu.async_copy(x_hbm.at[sidx_v], xs_v, sem)
    d2 = pltpu.async_copy(x_hbm.at[didx_v], xd_v, sem)
    d1.wait()
    d2.wait()

    lane17 = lax.iota(jnp.int32, 16) * 17

    def group_body(g, carry2):
      a16 = a_v[pl.ds(g * 16, 16)]
      for e16 in range(16):
        e = g * 16 + e16
        acc = jnp.zeros((16,), jnp.float32)
        for k in range(D // 16):
          dxy = xs_v[e, pl.ds(k * 16, 16)] - xd_v[e, pl.ds(k * 16, 16)]
          acc = acc + dxy * dxy
        tbuf[pl.ds(e16 * 17, 16)] = acc
      esum = jnp.zeros((16,), jnp.float32)
      for ccol in range(16):
        esum = esum + plsc.load_gather(tbuf, [lane17 + ccol])
      e_buf[pl.ds(g * 16, 16)] = a16 * esum
      return carry2

    lax.fori_loop(0, BLOCK // 16, group_body, 0)
    pltpu.sync_copy(e_buf, en_acc.at[didx_v], add=True)
    pltpu.sync_copy(f_v, deg_acc.at[didx_v], add=True)
    return carry

  lax.fori_loop(0, NBLK, block_body, 0)

  plsc.subcore_barrier()

  @pl.when(s == 0)
  def _writeout():
    pltpu.sync_copy(en_acc, en_out.at[c])
    pltpu.sync_copy(deg_acc, deg_out.at[c])


def _p1(x, srcp, dstp, ap, flagp, zeros_np):
  f32 = jnp.float32
  return pl.kernel(
      _p1_body,
      out_type=[
          jax.ShapeDtypeStruct((NC, N_PAD), f32),
          jax.ShapeDtypeStruct((NC, N_PAD), f32),
      ],
      mesh=_mesh,
      compiler_params=_sc_params,
      scratch_types=[
          pltpu.VMEM((BLOCK,), jnp.int32),
          pltpu.VMEM((BLOCK,), jnp.int32),
          pltpu.VMEM((BLOCK,), f32),
          pltpu.VMEM((BLOCK,), f32),
          pltpu.VMEM((BLOCK, D), f32),
          pltpu.VMEM((BLOCK, D), f32),
          pltpu.VMEM((BLOCK,), f32),
          pltpu.VMEM((16 * 17,), f32),
          pltpu.VMEM_SHARED((N_PAD,), f32),
          pltpu.VMEM_SHARED((N_PAD,), f32),
          pltpu.SemaphoreType.DMA,
      ],
  )(x, srcp, dstp, ap, flagp, zeros_np)


def _node_body(ep_ref, dp_ref, t_ref, w_ref, ne_ref, dinv_ref, cn_ref,
               dsq_ref):
  en = ep_ref[0] + ep_ref[1]
  deg = dp_ref[0] + dp_ref[1] + 1.0
  row = lax.broadcasted_iota(jnp.int32, (N_PAD // 128, 128), 0)
  col = lax.broadcasted_iota(jnp.int32, (N_PAD // 128, 128), 1)
  valid = (row * 128 + col) < N

  T = t_ref[0, 0]
  wgt = w_ref[0, 0]
  do_norm = ne_ref[0, 0] != 0
  neg_inf = jnp.float32(-jnp.inf)

  m = jnp.max(jnp.where(valid, en, neg_inf))
  r = 1.0 / (m + EPS)
  en_n = jnp.where(do_norm, en * r, en)

  z = -en_n / T
  zmax = jnp.max(jnp.where(valid, z, neg_inf))
  ez = jnp.where(valid, jnp.exp(z - zmax), 0.0)
  p = ez / jnp.sum(ez)
  g = -(jnp.log(p + EPS) + p / (p + EPS))
  pg = jnp.sum(jnp.where(valid, p * g, 0.0))
  u = (-1.0 / T) * p * (g - pg)
  sum_ue = jnp.sum(jnp.where(valid, u * en, 0.0))
  is_max = jnp.where(valid & (en == m), 1.0, 0.0)
  ties = jnp.sum(is_max)
  v = jnp.where(do_norm, u * r - (r * r) * sum_ue * is_max / ties, u)

  dinv = lax.rsqrt(deg)
  dinv_ref[...] = dinv
  cn_ref[...] = 2.0 * wgt * v
  dsq_ref[...] = 1.0 / deg


def _node_math(en_part, deg_part, temperature, weight, norm_energies):
  f32 = jnp.float32
  shp = (N_PAD // 128, 128)
  return pl.pallas_call(
      _node_body,
      in_specs=[
          pl.BlockSpec((NC,) + shp, lambda: (0, 0, 0)),
          pl.BlockSpec((NC,) + shp, lambda: (0, 0, 0)),
          pl.BlockSpec((1, 1), lambda: (0, 0)),
          pl.BlockSpec((1, 1), lambda: (0, 0)),
          pl.BlockSpec((1, 1), lambda: (0, 0)),
      ],
      out_specs=[
          pl.BlockSpec(shp, lambda: (0, 0)),
          pl.BlockSpec(shp, lambda: (0, 0)),
          pl.BlockSpec(shp, lambda: (0, 0)),
      ],
      out_shape=[
          jax.ShapeDtypeStruct(shp, f32),
          jax.ShapeDtypeStruct(shp, f32),
          jax.ShapeDtypeStruct(shp, f32),
      ],
  )(en_part.reshape((NC,) + shp), deg_part.reshape((NC,) + shp),
    temperature.reshape(1, 1), weight.reshape(1, 1),
    jnp.asarray(norm_energies, jnp.int32).reshape(1, 1))


def _p15_body(srcp, dstp, ap, flagp, dinv_hbm, cn_hbm, zeros_np,
              alpha_out, sval_out, s_out,
              sidx_v, didx_v, a_v, f_v, al_v, sv_v, dinv_t, cn_t,
              sacc):
  c = lax.axis_index("c")
  s = lax.axis_index("s")
  w = c * NS + s

  pltpu.sync_copy(dinv_hbm, dinv_t)
  pltpu.sync_copy(cn_hbm, cn_t)

  @pl.when(s == 0)
  def _init():
    pltpu.sync_copy(zeros_np, sacc)

  plsc.subcore_barrier()

  def block_body(b, carry):
    pltpu.sync_copy(srcp.at[w, b], sidx_v)
    pltpu.sync_copy(dstp.at[w, b], didx_v)
    pltpu.sync_copy(ap.at[w, b], a_v)
    pltpu.sync_copy(flagp.at[w, b], f_v)
    for k in range(BLOCK // 16):
      sl = pl.ds(k * 16, 16)
      idx_s = sidx_v[sl]
      idx_d = didx_v[sl]
      dv_s = plsc.load_gather(dinv_t, [idx_s])
      dv_d = plsc.load_gather(dinv_t, [idx_d])
      cn_d = plsc.load_gather(cn_t, [idx_d])
      al_v[sl] = f_v[sl] * dv_s * dv_d
      sv_v[sl] = a_v[sl] * cn_d
    pltpu.sync_copy(al_v, alpha_out.at[w, b])
    pltpu.sync_copy(sv_v, sval_out.at[w, b])
    pltpu.sync_copy(sv_v, sacc.at[sidx_v], add=True)
    pltpu.sync_copy(sv_v, sacc.at[didx_v], add=True)
    return carry

  lax.fori_loop(0, NBLK, block_body, 0)

  plsc.subcore_barrier()

  @pl.when(s == 0)
  def _writeout():
    pltpu.sync_copy(sacc, s_out.at[c])


def _p15(srcp, dstp, ap, flagp, dinv_n, cn_n, zeros_np):
  f32 = jnp.float32
  return pl.kernel(
      _p15_body,
      out_type=[
          jax.ShapeDtypeStruct((NW, NBLK, BLOCK), f32),
          jax.ShapeDtypeStruct((NW, NBLK, BLOCK), f32),
          jax.ShapeDtypeStruct((NC, N_PAD), f32),
      ],
      mesh=_mesh,
      compiler_params=_sc_params,
      scratch_types=[
          pltpu.VMEM((BLOCK,), jnp.int32),
          pltpu.VMEM((BLOCK,), jnp.int32),
          pltpu.VMEM((BLOCK,), f32),
          pltpu.VMEM((BLOCK,), f32),
          pltpu.VMEM((BLOCK,), f32),
          pltpu.VMEM((BLOCK,), f32),
          pltpu.VMEM((N,), f32),
          pltpu.VMEM((N,), f32),
          pltpu.VMEM_SHARED((N_PAD,), f32),
      ],
  )(srcp, dstp, ap, flagp, dinv_n, cn_n, zeros_np)


def _p2_body(xcat, hcat, srcp, dstp, alphap, svalp, zeros_nd,
             acc_out,
             sidx_v, didx_v, sidx2_v, didx2_v, al_v, sv_v,
             hs_v, xs_v,
             accum, sem):
  c = lax.axis_index("c")
  s = lax.axis_index("s")

  pltpu.sync_copy(zeros_nd.at[pl.ds(s * ROWS_ACC, ROWS_ACC)],
                  accum.at[pl.ds(s * ROWS_ACC, ROWS_ACC)])

  plsc.subcore_barrier()

  row_off = c * N

  def block_body(wj, b):
    pltpu.sync_copy(srcp.at[wj, b], sidx_v)
    pltpu.sync_copy(dstp.at[wj, b], didx_v)
    pltpu.sync_copy(alphap.at[wj, b], al_v)
    pltpu.sync_copy(svalp.at[wj, b], sv_v)

    for k in range(BLOCK // 16):
      sl = pl.ds(k * 16, 16)
      sidx2_v[sl] = sidx_v[sl] + row_off
      didx2_v[sl] = didx_v[sl] + row_off

    d1 = pltpu.async_copy(hcat.at[sidx2_v], hs_v, sem)
    d2 = pltpu.async_copy(xcat.at[sidx2_v], xs_v, sem)
    d1.wait()
    d2.wait()

    def group_body(g, carry2):
      a16 = al_v[pl.ds(g * 16, 16)]
      b16 = sv_v[pl.ds(g * 16, 16)]
      for e16 in range(16):
        e = g * 16 + e16
        ae = a16[e16]
        be = -b16[e16]
        for k in range(DH // 16):
          sl = pl.ds(k * 16, 16)
          xs_v[e, sl] = ae * hs_v[e, sl] + be * xs_v[e, sl]
      return carry2

    lax.fori_loop(0, BLOCK // 16, group_body, 0)
    pltpu.sync_copy(xs_v, accum.at[didx_v], add=True)

    pltpu.async_copy(xcat.at[didx2_v], hs_v, sem).wait()

    def group_body2(g, carry2):
      b16 = sv_v[pl.ds(g * 16, 16)]
      for e16 in range(16):
        e = g * 16 + e16
        be = -b16[e16]
        for k in range(DH // 16):
          sl = pl.ds(k * 16, 16)
          hs_v[e, sl] = be * hs_v[e, sl]
      return carry2

    lax.fori_loop(0, BLOCK // 16, group_body2, 0)
    pltpu.sync_copy(hs_v, accum.at[sidx_v], add=True)

  def outer(b, carry):
    block_body(2 * s, b)
    block_body(2 * s + 1, b)
    return carry

  lax.fori_loop(0, NBLK, outer, 0)

  plsc.subcore_barrier()

  rsl = pl.ds(s * ROWS_ACC, ROWS_ACC)
  pltpu.sync_copy(accum.at[rsl], acc_out.at[c, rsl])


def _p2(xcat, hcat, srcp, dstp, alphap, svalp, zeros_nd):
  f32 = jnp.float32
  i32 = jnp.int32
  return pl.kernel(
      _p2_body,
      out_type=jax.ShapeDtypeStruct((NC, N_ACC, DH), f32),
      mesh=_mesh,
      compiler_params=_sc_params,
      scratch_types=[
          pltpu.VMEM((BLOCK,), i32),
          pltpu.VMEM((BLOCK,), i32),
          pltpu.VMEM((BLOCK,), i32),
          pltpu.VMEM((BLOCK,), i32),
          pltpu.VMEM((BLOCK,), f32),
          pltpu.VMEM((BLOCK,), f32),
          pltpu.VMEM((BLOCK, DH), f32),
          pltpu.VMEM((BLOCK, DH), f32),
          pltpu.VMEM_SHARED((N_ACC, DH), f32),
          pltpu.SemaphoreType.DMA,
      ],
  )(xcat, hcat, srcp, dstp, alphap, svalp, zeros_nd)


def _combine_body(acc_ref, h_ref, x_ref, dsq_ref, s_ref, b_ref, o_ref):
  dsq = dsq_ref[...]
  sc = s_ref[0] + s_ref[1]
  bias = b_ref[...]
  o_ref[:, :DH] = (acc_ref[0] + dsq * h_ref[0] + sc * x_ref[:, :DH]
                   + bias[:, :DH])
  o_ref[:, DH:] = (acc_ref[1] + dsq * h_ref[1] + sc * x_ref[:, DH:]
                   + bias[:, DH:])


def _combine(acc, h3, x, dsq_col, s_part, b2d):
  RT = 400
  f32 = jnp.float32
  return pl.pallas_call(
      _combine_body,
      grid=(N // RT,),
      in_specs=[
          pl.BlockSpec((NC, RT, DH), lambda i: (0, i, 0)),
          pl.BlockSpec((NC, RT, DH), lambda i: (0, i, 0)),
          pl.BlockSpec((RT, D), lambda i: (i, 0)),
          pl.BlockSpec((RT, 1), lambda i: (i, 0)),
          pl.BlockSpec((NC, RT, 1), lambda i: (0, i, 0)),
          pl.BlockSpec((1, D), lambda i: (0, 0)),
      ],
      out_specs=pl.BlockSpec((RT, D), lambda i: (i, 0)),
      out_shape=jax.ShapeDtypeStruct((N, D), f32),
  )(acc, h3, x, dsq_col, s_part, b2d)


@jax.jit
def _run(x, edge_index, A, weight, temperature, norm_energies, W, b):
  f32 = jnp.float32
  i32 = jnp.int32
  src = edge_index[0]
  dst = edge_index[1]

  pad = E_PAD - E
  shp_e = (NW, NBLK, BLOCK)
  srcp = jnp.concatenate([src, jnp.zeros((pad,), i32)]).reshape(shp_e)
  dstp = jnp.concatenate([dst, jnp.zeros((pad,), i32)]).reshape(shp_e)
  ap = jnp.concatenate([A, jnp.zeros((pad,), f32)]).reshape(shp_e)
  flagp = jnp.concatenate([jnp.ones((E,), f32), jnp.zeros((pad,), f32)]
                          ).reshape(shp_e)
  zeros_np = jnp.zeros((N_PAD,), f32)
  zeros_nd = jnp.zeros((N_ACC, DH), f32)

  h3 = _matmul(x, W)
  hcat = h3.reshape(2 * N, DH)
  xcat = x.reshape(N, 2, DH).transpose(1, 0, 2).reshape(2 * N, DH)

  en_part, deg_part = _p1(x, srcp, dstp, ap, flagp, zeros_np)
  dinv2d, cn2d, dsq2d = _node_math(en_part, deg_part, temperature, weight,
                                   norm_energies)
  dinv_n = dinv2d.reshape(-1)[:N]
  cn_n = cn2d.reshape(-1)[:N]

  alphap, svalp, s_part = _p15(srcp, dstp, ap, flagp, dinv_n, cn_n, zeros_np)
  acc = _p2(xcat, hcat, srcp, dstp, alphap, svalp, zeros_nd)

  out = _combine(acc, h3, x, dsq2d.reshape(-1)[:N].reshape(N, 1),
                 s_part[:, :N].reshape(NC, N, 1), b.reshape(1, D))
  return out


def kernel(x, edge_index, A, weight, temperature, norm_energies, W, b):
  return _run(x, edge_index, A, weight, temperature, norm_energies, W, b)


# speedup vs baseline: 3.7558x; 1.1839x over previous
import jax
import jax.numpy as jnp
from jax import lax
from jax.experimental import pallas as pl
from jax.experimental.pallas import tpu as pltpu
from jax.experimental.pallas import tpu_sc as plsc

N = 10000
E = 160000
D = 256
DH = 128
EPS = 1e-12

NC = 2
NS = 16
NW = NC * NS

BLOCK = 128
EPT_PAD = 5120
NBLK = EPT_PAD // BLOCK
BD = 64
NBD = EPT_PAD // BD
E_PAD = EPT_PAD * NW
N_PAD = 10240
N_ACC = 10112
ROWS_ACC = N_ACC // NS

_mesh = plsc.VectorSubcoreMesh(
    core_axis_name="c", subcore_axis_name="s", num_cores=NC, num_subcores=NS)
_sc_params = pltpu.CompilerParams(needs_layout_passes=False)


def _matmul_body(x_ref, w_ref, o_ref):
  o_ref[0] = jnp.dot(x_ref[...], w_ref[...],
                     preferred_element_type=jnp.float32)


def _matmul(x, W):
  RT = 400
  return pl.pallas_call(
      _matmul_body,
      grid=(N // RT, 2),
      in_specs=[
          pl.BlockSpec((RT, D), lambda i, c: (i, 0)),
          pl.BlockSpec((D, DH), lambda i, c: (0, c)),
      ],
      out_specs=pl.BlockSpec((1, RT, DH), lambda i, c: (c, i, 0)),
      out_shape=jax.ShapeDtypeStruct((2, N, DH), jnp.float32),
  )(x, W)


def _p1_body(x_hbm, srcp, dstp, ap, flagp, zeros_np,
             en_out, deg_out,
             sidx0, didx0, a0, f0, xs0, xd0, eb0,
             sidx1, didx1, a1, f1, xs1, xd1, eb1,
             tbuf, en_acc, deg_acc, sem0, sem1):
  c = lax.axis_index("c")
  s = lax.axis_index("s")
  w = c * NS + s
  bufs = ((sidx0, didx0, a0, f0, xs0, xd0, eb0, sem0),
          (sidx1, didx1, a1, f1, xs1, xd1, eb1, sem1))

  @pl.when(s == 0)
  def _init():
    pltpu.sync_copy(zeros_np, en_acc)
    pltpu.sync_copy(zeros_np, deg_acc)

  plsc.subcore_barrier()

  def fetch(b, buf):
    sidx_v, didx_v, a_v, f_v, xs_v, xd_v, _, sem = buf
    pltpu.sync_copy(srcp.at[w, b], sidx_v)
    pltpu.sync_copy(dstp.at[w, b], didx_v)
    pltpu.sync_copy(ap.at[w, b], a_v)
    pltpu.sync_copy(flagp.at[w, b], f_v)
    pltpu.async_copy(x_hbm.at[sidx_v], xs_v, sem)
    pltpu.async_copy(x_hbm.at[didx_v], xd_v, sem)

  def process(buf):
    sidx_v, didx_v, a_v, f_v, xs_v, xd_v, e_buf, sem = buf
    pltpu.make_async_copy(x_hbm.at[sidx_v], xs_v, sem).wait()
    pltpu.make_async_copy(x_hbm.at[didx_v], xd_v, sem).wait()
    lane17 = lax.iota(jnp.int32, 16) * 17

    def group_body(g, carry2):
      a16 = a_v[pl.ds(g * 16, 16)]
      for e16 in range(16):
        e = g * 16 + e16
        acc = jnp.zeros((16,), jnp.float32)
        for k in range(D // 16):
          dxy = xs_v[e, pl.ds(k * 16, 16)] - xd_v[e, pl.ds(k * 16, 16)]
          acc = acc + dxy * dxy
        tbuf[pl.ds(e16 * 17, 16)] = acc
      esum = jnp.zeros((16,), jnp.float32)
      for ccol in range(16):
        esum = esum + plsc.load_gather(tbuf, [lane17 + ccol])
      e_buf[pl.ds(g * 16, 16)] = a16 * esum
      return carry2

    lax.fori_loop(0, BD // 16, group_body, 0)
    pltpu.sync_copy(e_buf, en_acc.at[didx_v], add=True)
    pltpu.sync_copy(f_v, deg_acc.at[didx_v], add=True)

  fetch(0, bufs[0])
  NG = NBD // 2

  def pair_body(g, carry):
    fetch(2 * g + 1, bufs[1])
    process(bufs[0])

    @pl.when(g < NG - 1)
    def _prefetch():
      fetch(2 * g + 2, bufs[0])

    process(bufs[1])
    return carry

  lax.fori_loop(0, NG, pair_body, 0)

  plsc.subcore_barrier()

  @pl.when(s == 0)
  def _writeout():
    pltpu.sync_copy(en_acc, en_out.at[c])
    pltpu.sync_copy(deg_acc, deg_out.at[c])


def _p1(x, srcp, dstp, ap, flagp, zeros_np):
  f32 = jnp.float32
  i32 = jnp.int32
  dbuf = [
      pltpu.VMEM((BD,), i32),
      pltpu.VMEM((BD,), i32),
      pltpu.VMEM((BD,), f32),
      pltpu.VMEM((BD,), f32),
      pltpu.VMEM((BD, D), f32),
      pltpu.VMEM((BD, D), f32),
      pltpu.VMEM((BD,), f32),
  ]
  return pl.kernel(
      _p1_body,
      out_type=[
          jax.ShapeDtypeStruct((NC, N_PAD), f32),
          jax.ShapeDtypeStruct((NC, N_PAD), f32),
      ],
      mesh=_mesh,
      compiler_params=_sc_params,
      scratch_types=dbuf + dbuf + [
          pltpu.VMEM((16 * 17,), f32),
          pltpu.VMEM_SHARED((N_PAD,), f32),
          pltpu.VMEM_SHARED((N_PAD,), f32),
          pltpu.SemaphoreType.DMA,
          pltpu.SemaphoreType.DMA,
      ],
  )(x, srcp, dstp, ap, flagp, zeros_np)


def _node_body(ep_ref, dp_ref, t_ref, w_ref, ne_ref, dinv_ref, cn_ref,
               dsq_ref):
  en = ep_ref[0] + ep_ref[1]
  deg = dp_ref[0] + dp_ref[1] + 1.0
  row = lax.broadcasted_iota(jnp.int32, (N_PAD // 128, 128), 0)
  col = lax.broadcasted_iota(jnp.int32, (N_PAD // 128, 128), 1)
  valid = (row * 128 + col) < N

  T = t_ref[0, 0]
  wgt = w_ref[0, 0]
  do_norm = ne_ref[0, 0] != 0
  neg_inf = jnp.float32(-jnp.inf)

  m = jnp.max(jnp.where(valid, en, neg_inf))
  r = 1.0 / (m + EPS)
  en_n = jnp.where(do_norm, en * r, en)

  z = -en_n / T
  zmax = jnp.max(jnp.where(valid, z, neg_inf))
  ez = jnp.where(valid, jnp.exp(z - zmax), 0.0)
  p = ez / jnp.sum(ez)
  g = -(jnp.log(p + EPS) + p / (p + EPS))
  pg = jnp.sum(jnp.where(valid, p * g, 0.0))
  u = (-1.0 / T) * p * (g - pg)
  sum_ue = jnp.sum(jnp.where(valid, u * en, 0.0))
  is_max = jnp.where(valid & (en == m), 1.0, 0.0)
  ties = jnp.sum(is_max)
  v = jnp.where(do_norm, u * r - (r * r) * sum_ue * is_max / ties, u)

  dinv = lax.rsqrt(deg)
  dinv_ref[...] = dinv
  cn_ref[...] = 2.0 * wgt * v
  dsq_ref[...] = 1.0 / deg


def _node_math(en_part, deg_part, temperature, weight, norm_energies):
  f32 = jnp.float32
  shp = (N_PAD // 128, 128)
  return pl.pallas_call(
      _node_body,
      in_specs=[
          pl.BlockSpec((NC,) + shp, lambda: (0, 0, 0)),
          pl.BlockSpec((NC,) + shp, lambda: (0, 0, 0)),
          pl.BlockSpec((1, 1), lambda: (0, 0)),
          pl.BlockSpec((1, 1), lambda: (0, 0)),
          pl.BlockSpec((1, 1), lambda: (0, 0)),
      ],
      out_specs=[
          pl.BlockSpec(shp, lambda: (0, 0)),
          pl.BlockSpec(shp, lambda: (0, 0)),
          pl.BlockSpec(shp, lambda: (0, 0)),
      ],
      out_shape=[
          jax.ShapeDtypeStruct(shp, f32),
          jax.ShapeDtypeStruct(shp, f32),
          jax.ShapeDtypeStruct(shp, f32),
      ],
  )(en_part.reshape((NC,) + shp), deg_part.reshape((NC,) + shp),
    temperature.reshape(1, 1), weight.reshape(1, 1),
    jnp.asarray(norm_energies, jnp.int32).reshape(1, 1))


def _p15_body(srcp, dstp, ap, flagp, dinv_hbm, cn_hbm, zeros_np,
              alpha_out, sval_out, s_out,
              sidx_v, didx_v, a_v, f_v, al_v, sv_v, dinv_t, cn_t,
              sacc):
  c = lax.axis_index("c")
  s = lax.axis_index("s")
  w = c * NS + s

  pltpu.sync_copy(dinv_hbm, dinv_t)
  pltpu.sync_copy(cn_hbm, cn_t)

  @pl.when(s == 0)
  def _init():
    pltpu.sync_copy(zeros_np, sacc)

  plsc.subcore_barrier()

  def block_body(b, carry):
    pltpu.sync_copy(srcp.at[w, b], sidx_v)
    pltpu.sync_copy(dstp.at[w, b], didx_v)
    pltpu.sync_copy(ap.at[w, b], a_v)
    pltpu.sync_copy(flagp.at[w, b], f_v)
    for k in range(BLOCK // 16):
      sl = pl.ds(k * 16, 16)
      idx_s = sidx_v[sl]
      idx_d = didx_v[sl]
      dv_s = plsc.load_gather(dinv_t, [idx_s])
      dv_d = plsc.load_gather(dinv_t, [idx_d])
      cn_d = plsc.load_gather(cn_t, [idx_d])
      al_v[sl] = f_v[sl] * dv_s * dv_d
      sv_v[sl] = a_v[sl] * cn_d
    pltpu.sync_copy(al_v, alpha_out.at[w, b])
    pltpu.sync_copy(sv_v, sval_out.at[w, b])
    pltpu.sync_copy(sv_v, sacc.at[sidx_v], add=True)
    pltpu.sync_copy(sv_v, sacc.at[didx_v], add=True)
    return carry

  lax.fori_loop(0, NBLK, block_body, 0)

  plsc.subcore_barrier()

  @pl.when(s == 0)
  def _writeout():
    pltpu.sync_copy(sacc, s_out.at[c])


def _p15(srcp, dstp, ap, flagp, dinv_n, cn_n, zeros_np):
  f32 = jnp.float32
  return pl.kernel(
      _p15_body,
      out_type=[
          jax.ShapeDtypeStruct((NW, NBLK, BLOCK), f32),
          jax.ShapeDtypeStruct((NW, NBLK, BLOCK), f32),
          jax.ShapeDtypeStruct((NC, N_PAD), f32),
      ],
      mesh=_mesh,
      compiler_params=_sc_params,
      scratch_types=[
          pltpu.VMEM((BLOCK,), jnp.int32),
          pltpu.VMEM((BLOCK,), jnp.int32),
          pltpu.VMEM((BLOCK,), f32),
          pltpu.VMEM((BLOCK,), f32),
          pltpu.VMEM((BLOCK,), f32),
          pltpu.VMEM((BLOCK,), f32),
          pltpu.VMEM((N,), f32),
          pltpu.VMEM((N,), f32),
          pltpu.VMEM_SHARED((N_PAD,), f32),
      ],
  )(srcp, dstp, ap, flagp, dinv_n, cn_n, zeros_np)


def _p2_body(xcat, hcat, srcp, dstp, alphap, svalp, zeros_nd,
             acc_out,
             sidx0, didx0, gidx0, al0, sv0, hs0, xs0,
             sidx1, didx1, gidx1, al1, sv1, hs1, xs1,
             accum, sem0, sem1):
  c = lax.axis_index("c")
  s = lax.axis_index("s")
  bufs = ((sidx0, didx0, gidx0, al0, sv0, hs0, xs0, sem0),
          (sidx1, didx1, gidx1, al1, sv1, hs1, xs1, sem1))

  pltpu.sync_copy(zeros_nd.at[pl.ds(s * ROWS_ACC, ROWS_ACC)],
                  accum.at[pl.ds(s * ROWS_ACC, ROWS_ACC)])

  plsc.subcore_barrier()

  row_off = c * N

  def wjb(vb):
    return 2 * s + vb // NBD, vb % NBD

  def fetch_a(vb, buf):
    sidx_v, didx_v, gidx_v, al_v, sv_v, hs_v, xs_v, sem = buf
    wj, b = wjb(vb)
    pltpu.sync_copy(srcp.at[wj, b], sidx_v)
    pltpu.sync_copy(dstp.at[wj, b], didx_v)
    pltpu.sync_copy(alphap.at[wj, b], al_v)
    pltpu.sync_copy(svalp.at[wj, b], sv_v)
    for k in range(BD // 16):
      sl = pl.ds(k * 16, 16)
      gidx_v[sl] = sidx_v[sl] + row_off
    pltpu.async_copy(hcat.at[gidx_v], hs_v, sem)
    pltpu.async_copy(xcat.at[gidx_v], xs_v, sem)

  def process_a(buf):
    sidx_v, didx_v, gidx_v, al_v, sv_v, hs_v, xs_v, sem = buf
    pltpu.make_async_copy(hcat.at[gidx_v], hs_v, sem).wait()
    pltpu.make_async_copy(xcat.at[gidx_v], xs_v, sem).wait()

    def group_body(g, carry2):
      a16 = al_v[pl.ds(g * 16, 16)]
      b16 = sv_v[pl.ds(g * 16, 16)]
      for e16 in range(16):
        e = g * 16 + e16
        ae = a16[e16]
        be = -b16[e16]
        for k in range(DH // 16):
          sl = pl.ds(k * 16, 16)
          xs_v[e, sl] = ae * hs_v[e, sl] + be * xs_v[e, sl]
      return carry2

    lax.fori_loop(0, BD // 16, group_body, 0)
    pltpu.sync_copy(xs_v, accum.at[didx_v], add=True)

  def fetch_b(vb, buf):
    sidx_v, didx_v, gidx_v, al_v, sv_v, hs_v, xs_v, sem = buf
    wj, b = wjb(vb)
    pltpu.sync_copy(srcp.at[wj, b], sidx_v)
    pltpu.sync_copy(dstp.at[wj, b], didx_v)
    pltpu.sync_copy(svalp.at[wj, b], sv_v)
    for k in range(BD // 16):
      sl = pl.ds(k * 16, 16)
      gidx_v[sl] = didx_v[sl] + row_off
    pltpu.async_copy(xcat.at[gidx_v], hs_v, sem)

  def process_b(buf):
    sidx_v, didx_v, gidx_v, al_v, sv_v, hs_v, xs_v, sem = buf
    pltpu.make_async_copy(xcat.at[gidx_v], hs_v, sem).wait()

    def group_body(g, carry2):
      b16 = sv_v[pl.ds(g * 16, 16)]
      for e16 in range(16):
        e = g * 16 + e16
        be = -b16[e16]
        for k in range(DH // 16):
          sl = pl.ds(k * 16, 16)
          hs_v[e, sl] = be * hs_v[e, sl]
      return carry2

    lax.fori_loop(0, BD // 16, group_body, 0)
    pltpu.sync_copy(hs_v, accum.at[sidx_v], add=True)

  NBT = 2 * NBD
  NG = NBT // 2

  def run_pass(fetch, process):
    fetch(0, bufs[0])

    def pair_body(g, carry):
      fetch(2 * g + 1, bufs[1])
      process(bufs[0])

      @pl.when(g < NG - 1)
      def _prefetch():
        fetch(2 * g + 2, bufs[0])

      process(bufs[1])
      return carry

    lax.fori_loop(0, NG, pair_body, 0)

  run_pass(fetch_a, process_a)
  run_pass(fetch_b, process_b)

  plsc.subcore_barrier()

  rsl = pl.ds(s * ROWS_ACC, ROWS_ACC)
  pltpu.sync_copy(accum.at[rsl], acc_out.at[c, rsl])


def _p2(xcat, hcat, srcp, dstp, alphap, svalp, zeros_nd):
  f32 = jnp.float32
  i32 = jnp.int32
  dbuf = [
      pltpu.VMEM((BD,), i32),
      pltpu.VMEM((BD,), i32),
      pltpu.VMEM((BD,), i32),
      pltpu.VMEM((BD,), f32),
      pltpu.VMEM((BD,), f32),
      pltpu.VMEM((BD, DH), f32),
      pltpu.VMEM((BD, DH), f32),
  ]
  return pl.kernel(
      _p2_body,
      out_type=jax.ShapeDtypeStruct((NC, N_ACC, DH), f32),
      mesh=_mesh,
      compiler_params=_sc_params,
      scratch_types=dbuf + dbuf + [
          pltpu.VMEM_SHARED((N_ACC, DH), f32),
          pltpu.SemaphoreType.DMA,
          pltpu.SemaphoreType.DMA,
      ],
  )(xcat, hcat, srcp, dstp, alphap, svalp, zeros_nd)


def _combine_body(acc_ref, h_ref, x_ref, dsq_ref, s_ref, b_ref, o_ref):
  dsq = dsq_ref[...]
  sc = s_ref[0] + s_ref[1]
  bias = b_ref[...]
  o_ref[:, :DH] = (acc_ref[0] + dsq * h_ref[0] + sc * x_ref[:, :DH]
                   + bias[:, :DH])
  o_ref[:, DH:] = (acc_ref[1] + dsq * h_ref[1] + sc * x_ref[:, DH:]
                   + bias[:, DH:])


def _combine(acc, h3, x, dsq_col, s_part, b2d):
  RT = 400
  f32 = jnp.float32
  return pl.pallas_call(
      _combine_body,
      grid=(N // RT,),
      in_specs=[
          pl.BlockSpec((NC, RT, DH), lambda i: (0, i, 0)),
          pl.BlockSpec((NC, RT, DH), lambda i: (0, i, 0)),
          pl.BlockSpec((RT, D), lambda i: (i, 0)),
          pl.BlockSpec((RT, 1), lambda i: (i, 0)),
          pl.BlockSpec((NC, RT, 1), lambda i: (0, i, 0)),
          pl.BlockSpec((1, D), lambda i: (0, 0)),
      ],
      out_specs=pl.BlockSpec((RT, D), lambda i: (i, 0)),
      out_shape=jax.ShapeDtypeStruct((N, D), f32),
  )(acc, h3, x, dsq_col, s_part, b2d)


@jax.jit
def _run(x, edge_index, A, weight, temperature, norm_energies, W, b):
  f32 = jnp.float32
  i32 = jnp.int32
  src = edge_index[0]
  dst = edge_index[1]

  pad = E_PAD - E
  shp128 = (NW, NBLK, BLOCK)
  shp64 = (NW, NBD, BD)
  src_f = jnp.concatenate([src, jnp.zeros((pad,), i32)])
  dst_f = jnp.concatenate([dst, jnp.zeros((pad,), i32)])
  a_f = jnp.concatenate([A, jnp.zeros((pad,), f32)])
  flag_f = jnp.concatenate([jnp.ones((E,), f32), jnp.zeros((pad,), f32)])
  zeros_np = jnp.zeros((N_PAD,), f32)
  zeros_nd = jnp.zeros((N_ACC, DH), f32)

  h3 = _matmul(x, W)
  hcat = h3.reshape(2 * N, DH)
  xcat = x.reshape(N, 2, DH).transpose(1, 0, 2).reshape(2 * N, DH)

  en_part, deg_part = _p1(x, src_f.reshape(shp64), dst_f.reshape(shp64),
                          a_f.reshape(shp64), flag_f.reshape(shp64),
                          zeros_np)
  dinv2d, cn2d, dsq2d = _node_math(en_part, deg_part, temperature, weight,
                                   norm_energies)
  dinv_n = dinv2d.reshape(-1)[:N]
  cn_n = cn2d.reshape(-1)[:N]

  alphap, svalp, s_part = _p15(src_f.reshape(shp128), dst_f.reshape(shp128),
                               a_f.reshape(shp128), flag_f.reshape(shp128),
                               dinv_n, cn_n, zeros_np)
  acc = _p2(xcat, hcat, src_f.reshape(shp64), dst_f.reshape(shp64),
            alphap.reshape(shp64), svalp.reshape(shp64), zeros_nd)

  out = _combine(acc, h3, x, dsq2d.reshape(-1)[:N].reshape(N, 1),
                 s_part[:, :N].reshape(NC, N, 1), b.reshape(1, D))
  return out


def kernel(x, edge_index, A, weight, temperature, norm_energies, W, b):
  return _run(x, edge_index, A, weight, temperature, norm_energies, W, b)


# speedup vs baseline: 4.1720x; 1.1108x over previous
import jax
import jax.numpy as jnp
from jax import lax
from jax.experimental import pallas as pl
from jax.experimental.pallas import tpu as pltpu
from jax.experimental.pallas import tpu_sc as plsc

N = 10000
E = 160000
D = 256
DH = 128
EPS = 1e-12

NC = 2
NS = 16
NW = NC * NS

BLOCK = 128
EPT_PAD = 5120
NBLK = EPT_PAD // BLOCK
BD = 64
NBD = EPT_PAD // BD
E_PAD = EPT_PAD * NW
N_PAD = 10240
N_ACC = 10112
ROWS_ACC = N_ACC // NS

_mesh = plsc.VectorSubcoreMesh(
    core_axis_name="c", subcore_axis_name="s", num_cores=NC, num_subcores=NS)
_sc_params = pltpu.CompilerParams(needs_layout_passes=False)


def _matmul_body(x_ref, w_ref, o_ref):
  o_ref[0] = jnp.dot(x_ref[...], w_ref[...],
                     preferred_element_type=jnp.float32)


def _matmul(x, W):
  RT = 400
  return pl.pallas_call(
      _matmul_body,
      grid=(N // RT, 2),
      in_specs=[
          pl.BlockSpec((RT, D), lambda i, c: (i, 0)),
          pl.BlockSpec((D, DH), lambda i, c: (0, c)),
      ],
      out_specs=pl.BlockSpec((1, RT, DH), lambda i, c: (c, i, 0)),
      out_shape=jax.ShapeDtypeStruct((2, N, DH), jnp.float32),
  )(x, W)


def _p1_body(x_hbm, pk1, zeros_np,
             en_out, deg_out,
             pk0_v, xs0, xd0, eb0, fb0,
             pk1_v, xs1, xd1, eb1, fb1,
             tbuf, en_acc, deg_acc, sem0, sem1):
  c = lax.axis_index("c")
  s = lax.axis_index("s")
  w = c * NS + s
  bufs = ((pk0_v, xs0, xd0, eb0, fb0, sem0),
          (pk1_v, xs1, xd1, eb1, fb1, sem1))

  @pl.when(s == 0)
  def _init():
    pltpu.sync_copy(zeros_np, en_acc)
    pltpu.sync_copy(zeros_np, deg_acc)

  plsc.subcore_barrier()

  def fetch(b, buf):
    pk_v, xs_v, xd_v, _, _, sem = buf
    pltpu.sync_copy(pk1.at[w, b], pk_v)
    pltpu.async_copy(x_hbm.at[pk_v.at[0]], xs_v, sem)
    pltpu.async_copy(x_hbm.at[pk_v.at[1]], xd_v, sem)

  def process(buf):
    pk_v, xs_v, xd_v, e_buf, f_buf, sem = buf
    pltpu.make_async_copy(x_hbm.at[pk_v.at[0]], xs_v, sem).wait()
    pltpu.make_async_copy(x_hbm.at[pk_v.at[1]], xd_v, sem).wait()
    lane17 = lax.iota(jnp.int32, 16) * 17

    def group_body(g, carry2):
      sl = pl.ds(g * 16, 16)
      a16 = plsc.bitcast(pk_v[2, sl], jnp.float32)
      f_buf[sl] = plsc.bitcast(pk_v[3, sl], jnp.float32)
      for e16 in range(16):
        e = g * 16 + e16
        acc = jnp.zeros((16,), jnp.float32)
        for k in range(D // 16):
          dxy = xs_v[e, pl.ds(k * 16, 16)] - xd_v[e, pl.ds(k * 16, 16)]
          acc = acc + dxy * dxy
        tbuf[pl.ds(e16 * 17, 16)] = acc
      esum = jnp.zeros((16,), jnp.float32)
      for ccol in range(16):
        esum = esum + plsc.load_gather(tbuf, [lane17 + ccol])
      e_buf[sl] = a16 * esum
      return carry2

    lax.fori_loop(0, BD // 16, group_body, 0)
    pltpu.sync_copy(e_buf, en_acc.at[pk_v.at[1]], add=True)
    pltpu.sync_copy(f_buf, deg_acc.at[pk_v.at[1]], add=True)

  fetch(0, bufs[0])
  NG = NBD // 2

  def pair_body(g, carry):
    fetch(2 * g + 1, bufs[1])
    process(bufs[0])

    @pl.when(g < NG - 1)
    def _prefetch():
      fetch(2 * g + 2, bufs[0])

    process(bufs[1])
    return carry

  lax.fori_loop(0, NG, pair_body, 0)

  plsc.subcore_barrier()

  @pl.when(s == 0)
  def _writeout():
    pltpu.sync_copy(en_acc, en_out.at[c])
    pltpu.sync_copy(deg_acc, deg_out.at[c])


def _p1(x, pk1, zeros_np):
  f32 = jnp.float32
  i32 = jnp.int32
  dbuf = [
      pltpu.VMEM((4, BD), i32),
      pltpu.VMEM((BD, D), f32),
      pltpu.VMEM((BD, D), f32),
      pltpu.VMEM((BD,), f32),
      pltpu.VMEM((BD,), f32),
  ]
  return pl.kernel(
      _p1_body,
      out_type=[
          jax.ShapeDtypeStruct((NC, N_PAD), f32),
          jax.ShapeDtypeStruct((NC, N_PAD), f32),
      ],
      mesh=_mesh,
      compiler_params=_sc_params,
      scratch_types=dbuf + dbuf + [
          pltpu.VMEM((16 * 17,), f32),
          pltpu.VMEM_SHARED((N_PAD,), f32),
          pltpu.VMEM_SHARED((N_PAD,), f32),
          pltpu.SemaphoreType.DMA,
          pltpu.SemaphoreType.DMA,
      ],
  )(x, pk1, zeros_np)


def _node_body(ep_ref, dp_ref, t_ref, w_ref, ne_ref, dinv_ref, cn_ref,
               dsq_ref):
  en = ep_ref[0] + ep_ref[1]
  deg = dp_ref[0] + dp_ref[1] + 1.0
  row = lax.broadcasted_iota(jnp.int32, (N_PAD // 128, 128), 0)
  col = lax.broadcasted_iota(jnp.int32, (N_PAD // 128, 128), 1)
  valid = (row * 128 + col) < N

  T = t_ref[0, 0]
  wgt = w_ref[0, 0]
  do_norm = ne_ref[0, 0] != 0
  neg_inf = jnp.float32(-jnp.inf)

  m = jnp.max(jnp.where(valid, en, neg_inf))
  r = 1.0 / (m + EPS)
  en_n = jnp.where(do_norm, en * r, en)

  z = -en_n / T
  zmax = jnp.max(jnp.where(valid, z, neg_inf))
  ez = jnp.where(valid, jnp.exp(z - zmax), 0.0)
  p = ez / jnp.sum(ez)
  g = -(jnp.log(p + EPS) + p / (p + EPS))
  pg = jnp.sum(jnp.where(valid, p * g, 0.0))
  u = (-1.0 / T) * p * (g - pg)
  sum_ue = jnp.sum(jnp.where(valid, u * en, 0.0))
  is_max = jnp.where(valid & (en == m), 1.0, 0.0)
  ties = jnp.sum(is_max)
  v = jnp.where(do_norm, u * r - (r * r) * sum_ue * is_max / ties, u)

  dinv = lax.rsqrt(deg)
  dinv_ref[...] = dinv
  cn_ref[...] = 2.0 * wgt * v
  dsq_ref[...] = 1.0 / deg


def _node_math(en_part, deg_part, temperature, weight, norm_energies):
  f32 = jnp.float32
  shp = (N_PAD // 128, 128)
  return pl.pallas_call(
      _node_body,
      in_specs=[
          pl.BlockSpec((NC,) + shp, lambda: (0, 0, 0)),
          pl.BlockSpec((NC,) + shp, lambda: (0, 0, 0)),
          pl.BlockSpec((1, 1), lambda: (0, 0)),
          pl.BlockSpec((1, 1), lambda: (0, 0)),
          pl.BlockSpec((1, 1), lambda: (0, 0)),
      ],
      out_specs=[
          pl.BlockSpec(shp, lambda: (0, 0)),
          pl.BlockSpec(shp, lambda: (0, 0)),
          pl.BlockSpec(shp, lambda: (0, 0)),
      ],
      out_shape=[
          jax.ShapeDtypeStruct(shp, f32),
          jax.ShapeDtypeStruct(shp, f32),
          jax.ShapeDtypeStruct(shp, f32),
      ],
  )(en_part.reshape((NC,) + shp), deg_part.reshape((NC,) + shp),
    temperature.reshape(1, 1), weight.reshape(1, 1),
    jnp.asarray(norm_energies, jnp.int32).reshape(1, 1))


def _p15_body(srcp, dstp, ap, flagp, dinv_hbm, cn_hbm, zeros_np,
              alpha_out, sval_out, s_out,
              sidx_v, didx_v, a_v, f_v, al_v, sv_v, dinv_t, cn_t,
              sacc):
  c = lax.axis_index("c")
  s = lax.axis_index("s")
  w = c * NS + s

  pltpu.sync_copy(dinv_hbm, dinv_t)
  pltpu.sync_copy(cn_hbm, cn_t)

  @pl.when(s == 0)
  def _init():
    pltpu.sync_copy(zeros_np, sacc)

  plsc.subcore_barrier()

  def block_body(b, carry):
    pltpu.sync_copy(srcp.at[w, b], sidx_v)
    pltpu.sync_copy(dstp.at[w, b], didx_v)
    pltpu.sync_copy(ap.at[w, b], a_v)
    pltpu.sync_copy(flagp.at[w, b], f_v)
    for k in range(BLOCK // 16):
      sl = pl.ds(k * 16, 16)
      idx_s = sidx_v[sl]
      idx_d = didx_v[sl]
      dv_s = plsc.load_gather(dinv_t, [idx_s])
      dv_d = plsc.load_gather(dinv_t, [idx_d])
      cn_d = plsc.load_gather(cn_t, [idx_d])
      al_v[sl] = f_v[sl] * dv_s * dv_d
      sv_v[sl] = a_v[sl] * cn_d
    pltpu.sync_copy(al_v, alpha_out.at[w, b])
    pltpu.sync_copy(sv_v, sval_out.at[w, b])
    pltpu.sync_copy(sv_v, sacc.at[sidx_v], add=True)
    pltpu.sync_copy(sv_v, sacc.at[didx_v], add=True)
    return carry

  lax.fori_loop(0, NBLK, block_body, 0)

  plsc.subcore_barrier()

  @pl.when(s == 0)
  def _writeout():
    pltpu.sync_copy(sacc, s_out.at[c])


def _p15(srcp, dstp, ap, flagp, dinv_n, cn_n, zeros_np):
  f32 = jnp.float32
  return pl.kernel(
      _p15_body,
      out_type=[
          jax.ShapeDtypeStruct((NW, NBLK, BLOCK), f32),
          jax.ShapeDtypeStruct((NW, NBLK, BLOCK), f32),
          jax.ShapeDtypeStruct((NC, N_PAD), f32),
      ],
      mesh=_mesh,
      compiler_params=_sc_params,
      scratch_types=[
          pltpu.VMEM((BLOCK,), jnp.int32),
          pltpu.VMEM((BLOCK,), jnp.int32),
          pltpu.VMEM((BLOCK,), f32),
          pltpu.VMEM((BLOCK,), f32),
          pltpu.VMEM((BLOCK,), f32),
          pltpu.VMEM((BLOCK,), f32),
          pltpu.VMEM((N,), f32),
          pltpu.VMEM((N,), f32),
          pltpu.VMEM_SHARED((N_PAD,), f32),
      ],
  )(srcp, dstp, ap, flagp, dinv_n, cn_n, zeros_np)


def _p2_body(xcat, hcat, pk2, zeros_nd,
             acc_out,
             pk0_v, gidx0, hs0, xs0,
             pk1_v, gidx1, hs1, xs1,
             accum, sem0, sem1):
  c = lax.axis_index("c")
  s = lax.axis_index("s")
  bufs = ((pk0_v, gidx0, hs0, xs0, sem0),
          (pk1_v, gidx1, hs1, xs1, sem1))

  pltpu.sync_copy(zeros_nd.at[pl.ds(s * ROWS_ACC, ROWS_ACC)],
                  accum.at[pl.ds(s * ROWS_ACC, ROWS_ACC)])

  plsc.subcore_barrier()

  row_off = c * N

  def wjb(vb):
    return 2 * s + vb // NBD, vb % NBD

  def fetch_a(vb, buf):
    pk_v, gidx_v, hs_v, xs_v, sem = buf
    wj, b = wjb(vb)
    pltpu.sync_copy(pk2.at[wj, b], pk_v)
    for k in range(BD // 16):
      sl = pl.ds(k * 16, 16)
      gidx_v[sl] = pk_v[0, sl] + row_off
    pltpu.async_copy(hcat.at[gidx_v], hs_v, sem)
    pltpu.async_copy(xcat.at[gidx_v], xs_v, sem)

  def process_a(buf):
    pk_v, gidx_v, hs_v, xs_v, sem = buf
    pltpu.make_async_copy(hcat.at[gidx_v], hs_v, sem).wait()
    pltpu.make_async_copy(xcat.at[gidx_v], xs_v, sem).wait()

    def group_body(g, carry2):
      gsl = pl.ds(g * 16, 16)
      a16 = plsc.bitcast(pk_v[2, gsl], jnp.float32)
      b16 = plsc.bitcast(pk_v[3, gsl], jnp.float32)
      for e16 in range(16):
        e = g * 16 + e16
        ae = a16[e16]
        be = -b16[e16]
        for k in range(DH // 16):
          sl = pl.ds(k * 16, 16)
          xs_v[e, sl] = ae * hs_v[e, sl] + be * xs_v[e, sl]
      return carry2

    lax.fori_loop(0, BD // 16, group_body, 0)
    pltpu.sync_copy(xs_v, accum.at[pk_v.at[1]], add=True)

  def fetch_b(vb, buf):
    pk_v, gidx_v, hs_v, xs_v, sem = buf
    wj, b = wjb(vb)
    pltpu.sync_copy(pk2.at[wj, b], pk_v)
    for k in range(BD // 16):
      sl = pl.ds(k * 16, 16)
      gidx_v[sl] = pk_v[1, sl] + row_off
    pltpu.async_copy(xcat.at[gidx_v], hs_v, sem)

  def process_b(buf):
    pk_v, gidx_v, hs_v, xs_v, sem = buf
    pltpu.make_async_copy(xcat.at[gidx_v], hs_v, sem).wait()

    def group_body(g, carry2):
      b16 = plsc.bitcast(pk_v[3, pl.ds(g * 16, 16)], jnp.float32)
      for e16 in range(16):
        e = g * 16 + e16
        be = -b16[e16]
        for k in range(DH // 16):
          sl = pl.ds(k * 16, 16)
          hs_v[e, sl] = be * hs_v[e, sl]
      return carry2

    lax.fori_loop(0, BD // 16, group_body, 0)
    pltpu.sync_copy(hs_v, accum.at[pk_v.at[0]], add=True)

  NBT = 2 * NBD
  NG = NBT // 2

  def run_pass(fetch, process):
    fetch(0, bufs[0])

    def pair_body(g, carry):
      fetch(2 * g + 1, bufs[1])
      process(bufs[0])

      @pl.when(g < NG - 1)
      def _prefetch():
        fetch(2 * g + 2, bufs[0])

      process(bufs[1])
      return carry

    lax.fori_loop(0, NG, pair_body, 0)

  run_pass(fetch_a, process_a)
  run_pass(fetch_b, process_b)

  plsc.subcore_barrier()

  rsl = pl.ds(s * ROWS_ACC, ROWS_ACC)
  pltpu.sync_copy(accum.at[rsl], acc_out.at[c, rsl])


def _p2(xcat, hcat, pk2, zeros_nd):
  f32 = jnp.float32
  i32 = jnp.int32
  dbuf = [
      pltpu.VMEM((4, BD), i32),
      pltpu.VMEM((BD,), i32),
      pltpu.VMEM((BD, DH), f32),
      pltpu.VMEM((BD, DH), f32),
  ]
  return pl.kernel(
      _p2_body,
      out_type=jax.ShapeDtypeStruct((NC, N_ACC, DH), f32),
      mesh=_mesh,
      compiler_params=_sc_params,
      scratch_types=dbuf + dbuf + [
          pltpu.VMEM_SHARED((N_ACC, DH), f32),
          pltpu.SemaphoreType.DMA,
          pltpu.SemaphoreType.DMA,
      ],
  )(xcat, hcat, pk2, zeros_nd)


def _combine_body(acc_ref, h_ref, x_ref, dsq_ref, s_ref, b_ref, o_ref):
  dsq = dsq_ref[...]
  sc = s_ref[0] + s_ref[1]
  bias = b_ref[...]
  o_ref[:, :DH] = (acc_ref[0] + dsq * h_ref[0] + sc * x_ref[:, :DH]
                   + bias[:, :DH])
  o_ref[:, DH:] = (acc_ref[1] + dsq * h_ref[1] + sc * x_ref[:, DH:]
                   + bias[:, DH:])


def _combine(acc, h3, x, dsq_col, s_part, b2d):
  RT = 400
  f32 = jnp.float32
  return pl.pallas_call(
      _combine_body,
      grid=(N // RT,),
      in_specs=[
          pl.BlockSpec((NC, RT, DH), lambda i: (0, i, 0)),
          pl.BlockSpec((NC, RT, DH), lambda i: (0, i, 0)),
          pl.BlockSpec((RT, D), lambda i: (i, 0)),
          pl.BlockSpec((RT, 1), lambda i: (i, 0)),
          pl.BlockSpec((NC, RT, 1), lambda i: (0, i, 0)),
          pl.BlockSpec((1, D), lambda i: (0, 0)),
      ],
      out_specs=pl.BlockSpec((RT, D), lambda i: (i, 0)),
      out_shape=jax.ShapeDtypeStruct((N, D), f32),
  )(acc, h3, x, dsq_col, s_part, b2d)


@jax.jit
def _run(x, edge_index, A, weight, temperature, norm_energies, W, b):
  f32 = jnp.float32
  i32 = jnp.int32
  src = edge_index[0]
  dst = edge_index[1]

  pad = E_PAD - E
  shp128 = (NW, NBLK, BLOCK)
  shp64 = (NW, NBD, BD)
  src_f = jnp.concatenate([src, jnp.zeros((pad,), i32)])
  dst_f = jnp.concatenate([dst, jnp.zeros((pad,), i32)])
  a_f = jnp.concatenate([A, jnp.zeros((pad,), f32)])
  flag_f = jnp.concatenate([jnp.ones((E,), f32), jnp.zeros((pad,), f32)])
  zeros_np = jnp.zeros((N_PAD,), f32)
  zeros_nd = jnp.zeros((N_ACC, DH), f32)

  h3 = _matmul(x, W)
  hcat = h3.reshape(2 * N, DH)
  xcat = x.reshape(N, 2, DH).transpose(1, 0, 2).reshape(2 * N, DH)

  pk1 = jnp.stack(
      [src_f.reshape(shp64), dst_f.reshape(shp64),
       lax.bitcast_convert_type(a_f, i32).reshape(shp64),
       lax.bitcast_convert_type(flag_f, i32).reshape(shp64)], axis=2)

  en_part, deg_part = _p1(x, pk1, zeros_np)
  dinv2d, cn2d, dsq2d = _node_math(en_part, deg_part, temperature, weight,
                                   norm_energies)
  dinv_n = dinv2d.reshape(-1)[:N]
  cn_n = cn2d.reshape(-1)[:N]

  alphap, svalp, s_part = _p15(src_f.reshape(shp128), dst_f.reshape(shp128),
                               a_f.reshape(shp128), flag_f.reshape(shp128),
                               dinv_n, cn_n, zeros_np)
  pk2 = jnp.stack(
      [src_f.reshape(shp64), dst_f.reshape(shp64),
       lax.bitcast_convert_type(alphap, i32).reshape(shp64),
       lax.bitcast_convert_type(svalp, i32).reshape(shp64)], axis=2)
  acc = _p2(xcat, hcat, pk2, zeros_nd)

  out = _combine(acc, h3, x, dsq2d.reshape(-1)[:N].reshape(N, 1),
                 s_part[:, :N].reshape(NC, N, 1), b.reshape(1, D))
  return out


def kernel(x, edge_index, A, weight, temperature, norm_energies, W, b):
  return _run(x, edge_index, A, weight, temperature, norm_energies, W, b)


# speedup vs baseline: 4.1778x; 1.0014x over previous
import jax
import jax.numpy as jnp
from jax import lax
from jax.experimental import pallas as pl
from jax.experimental.pallas import tpu as pltpu
from jax.experimental.pallas import tpu_sc as plsc

N = 10000
E = 160000
D = 256
DH = 128
EPS = 1e-12

NC = 2
NS = 16
NW = NC * NS

BLOCK = 128
EPT_PAD = 5120
NBLK = EPT_PAD // BLOCK
BD = 64
NBD = EPT_PAD // BD
E_PAD = EPT_PAD * NW
N_PAD = 10240
N_ACC = 10112
ROWS_ACC = N_ACC // NS

_mesh = plsc.VectorSubcoreMesh(
    core_axis_name="c", subcore_axis_name="s", num_cores=NC, num_subcores=NS)
_sc_params = pltpu.CompilerParams(needs_layout_passes=False)


def _matmul_body(x_ref, w_ref, o_ref):
  o_ref[0] = jnp.dot(x_ref[...], w_ref[...],
                     preferred_element_type=jnp.float32)


def _matmul(x, W):
  RT = 400
  return pl.pallas_call(
      _matmul_body,
      grid=(N // RT, 2),
      in_specs=[
          pl.BlockSpec((RT, D), lambda i, c: (i, 0)),
          pl.BlockSpec((D, DH), lambda i, c: (0, c)),
      ],
      out_specs=pl.BlockSpec((1, RT, DH), lambda i, c: (c, i, 0)),
      out_shape=jax.ShapeDtypeStruct((2, N, DH), jnp.float32),
  )(x, W)


def _p1_body(x_hbm, pk1, zeros_np,
             en_out, deg_out,
             pk0_v, xs0, xd0, eb0, fb0,
             pk1_v, xs1, xd1, eb1, fb1,
             tbuf, en_acc, deg_acc, sem0, sem1):
  c = lax.axis_index("c")
  s = lax.axis_index("s")
  w = c * NS + s
  bufs = ((pk0_v, xs0, xd0, eb0, fb0, sem0),
          (pk1_v, xs1, xd1, eb1, fb1, sem1))

  @pl.when(s == 0)
  def _init():
    pltpu.sync_copy(zeros_np, en_acc)
    pltpu.sync_copy(zeros_np, deg_acc)

  plsc.subcore_barrier()

  def fetch(b, buf):
    pk_v, xs_v, xd_v, _, _, sem = buf
    pltpu.sync_copy(pk1.at[w, b], pk_v)
    pltpu.async_copy(x_hbm.at[pk_v.at[0]], xs_v, sem)
    pltpu.async_copy(x_hbm.at[pk_v.at[1]], xd_v, sem)

  def process(buf):
    pk_v, xs_v, xd_v, e_buf, f_buf, sem = buf
    pltpu.make_async_copy(x_hbm.at[pk_v.at[0]], xs_v, sem).wait()
    pltpu.make_async_copy(x_hbm.at[pk_v.at[1]], xd_v, sem).wait()
    lane17 = lax.iota(jnp.int32, 16) * 17

    def group_body(g, carry2):
      sl = pl.ds(g * 16, 16)
      a16 = plsc.bitcast(pk_v[2, sl], jnp.float32)
      f_buf[sl] = plsc.bitcast(pk_v[3, sl], jnp.float32)
      for e16 in range(16):
        e = g * 16 + e16
        acc = jnp.zeros((16,), jnp.float32)
        for k in range(D // 16):
          dxy = xs_v[e, pl.ds(k * 16, 16)] - xd_v[e, pl.ds(k * 16, 16)]
          acc = acc + dxy * dxy
        tbuf[pl.ds(e16 * 17, 16)] = acc
      esum = jnp.zeros((16,), jnp.float32)
      for ccol in range(16):
        esum = esum + plsc.load_gather(tbuf, [lane17 + ccol])
      e_buf[sl] = a16 * esum
      return carry2

    lax.fori_loop(0, BD // 16, group_body, 0)
    pltpu.sync_copy(e_buf, en_acc.at[pk_v.at[1]], add=True)
    pltpu.sync_copy(f_buf, deg_acc.at[pk_v.at[1]], add=True)

  fetch(0, bufs[0])
  NG = NBD // 2

  def pair_body(g, carry):
    fetch(2 * g + 1, bufs[1])
    process(bufs[0])

    @pl.when(g < NG - 1)
    def _prefetch():
      fetch(2 * g + 2, bufs[0])

    process(bufs[1])
    return carry

  lax.fori_loop(0, NG, pair_body, 0)

  plsc.subcore_barrier()

  @pl.when(s == 0)
  def _writeout():
    pltpu.sync_copy(en_acc, en_out.at[c])
    pltpu.sync_copy(deg_acc, deg_out.at[c])


def _p1(x, pk1, zeros_np):
  f32 = jnp.float32
  i32 = jnp.int32
  dbuf = [
      pltpu.VMEM((4, BD), i32),
      pltpu.VMEM((BD, D), f32),
      pltpu.VMEM((BD, D), f32),
      pltpu.VMEM((BD,), f32),
      pltpu.VMEM((BD,), f32),
  ]
  return pl.kernel(
      _p1_body,
      out_type=[
          jax.ShapeDtypeStruct((NC, N_PAD), f32),
          jax.ShapeDtypeStruct((NC, N_PAD), f32),
      ],
      mesh=_mesh,
      compiler_params=_sc_params,
      scratch_types=dbuf + dbuf + [
          pltpu.VMEM((16 * 17,), f32),
          pltpu.VMEM_SHARED((N_PAD,), f32),
          pltpu.VMEM_SHARED((N_PAD,), f32),
          pltpu.SemaphoreType.DMA,
          pltpu.SemaphoreType.DMA,
      ],
  )(x, pk1, zeros_np)


def _node_body(ep_ref, dp_ref, t_ref, w_ref, ne_ref, dinv_ref, cn_ref,
               dsq_ref):
  en = ep_ref[0] + ep_ref[1]
  deg = dp_ref[0] + dp_ref[1] + 1.0
  row = lax.broadcasted_iota(jnp.int32, (N_PAD // 128, 128), 0)
  col = lax.broadcasted_iota(jnp.int32, (N_PAD // 128, 128), 1)
  valid = (row * 128 + col) < N

  T = t_ref[0, 0]
  wgt = w_ref[0, 0]
  do_norm = ne_ref[0, 0] != 0
  neg_inf = jnp.float32(-jnp.inf)

  m = jnp.max(jnp.where(valid, en, neg_inf))
  r = 1.0 / (m + EPS)
  en_n = jnp.where(do_norm, en * r, en)

  z = -en_n / T
  zmax = jnp.max(jnp.where(valid, z, neg_inf))
  ez = jnp.where(valid, jnp.exp(z - zmax), 0.0)
  p = ez / jnp.sum(ez)
  g = -(jnp.log(p + EPS) + p / (p + EPS))
  pg = jnp.sum(jnp.where(valid, p * g, 0.0))
  u = (-1.0 / T) * p * (g - pg)
  sum_ue = jnp.sum(jnp.where(valid, u * en, 0.0))
  is_max = jnp.where(valid & (en == m), 1.0, 0.0)
  ties = jnp.sum(is_max)
  v = jnp.where(do_norm, u * r - (r * r) * sum_ue * is_max / ties, u)

  dinv = lax.rsqrt(deg)
  dinv_ref[...] = dinv
  cn_ref[...] = 2.0 * wgt * v
  dsq_ref[...] = 1.0 / deg


def _node_math(en_part, deg_part, temperature, weight, norm_energies):
  f32 = jnp.float32
  shp = (N_PAD // 128, 128)
  return pl.pallas_call(
      _node_body,
      in_specs=[
          pl.BlockSpec((NC,) + shp, lambda: (0, 0, 0)),
          pl.BlockSpec((NC,) + shp, lambda: (0, 0, 0)),
          pl.BlockSpec((1, 1), lambda: (0, 0)),
          pl.BlockSpec((1, 1), lambda: (0, 0)),
          pl.BlockSpec((1, 1), lambda: (0, 0)),
      ],
      out_specs=[
          pl.BlockSpec(shp, lambda: (0, 0)),
          pl.BlockSpec(shp, lambda: (0, 0)),
          pl.BlockSpec(shp, lambda: (0, 0)),
      ],
      out_shape=[
          jax.ShapeDtypeStruct(shp, f32),
          jax.ShapeDtypeStruct(shp, f32),
          jax.ShapeDtypeStruct(shp, f32),
      ],
  )(en_part.reshape((NC,) + shp), deg_part.reshape((NC,) + shp),
    temperature.reshape(1, 1), weight.reshape(1, 1),
    jnp.asarray(norm_energies, jnp.int32).reshape(1, 1))


def _p15_body(srcp, dstp, ap, flagp, dinv_hbm, cn_hbm, zeros_np,
              alpha_out, sval_out, s_out,
              sidx_v, didx_v, a_v, f_v, al_v, sv_v, dinv_t, cn_t,
              sacc):
  c = lax.axis_index("c")
  s = lax.axis_index("s")
  w = c * NS + s

  pltpu.sync_copy(dinv_hbm, dinv_t)
  pltpu.sync_copy(cn_hbm, cn_t)

  @pl.when(s == 0)
  def _init():
    pltpu.sync_copy(zeros_np, sacc)

  plsc.subcore_barrier()

  def block_body(b, carry):
    pltpu.sync_copy(srcp.at[w, b], sidx_v)
    pltpu.sync_copy(dstp.at[w, b], didx_v)
    pltpu.sync_copy(ap.at[w, b], a_v)
    pltpu.sync_copy(flagp.at[w, b], f_v)
    for k in range(BLOCK // 16):
      sl = pl.ds(k * 16, 16)
      idx_s = sidx_v[sl]
      idx_d = didx_v[sl]
      dv_s = plsc.load_gather(dinv_t, [idx_s])
      dv_d = plsc.load_gather(dinv_t, [idx_d])
      cn_d = plsc.load_gather(cn_t, [idx_d])
      al_v[sl] = f_v[sl] * dv_s * dv_d
      sv_v[sl] = a_v[sl] * cn_d
    pltpu.sync_copy(al_v, alpha_out.at[w, b])
    pltpu.sync_copy(sv_v, sval_out.at[w, b])
    pltpu.sync_copy(sv_v, sacc.at[sidx_v], add=True)
    pltpu.sync_copy(sv_v, sacc.at[didx_v], add=True)
    return carry

  lax.fori_loop(0, NBLK, block_body, 0)

  plsc.subcore_barrier()

  @pl.when(s == 0)
  def _writeout():
    pltpu.sync_copy(sacc, s_out.at[c])


def _p15(srcp, dstp, ap, flagp, dinv_n, cn_n, zeros_np):
  f32 = jnp.float32
  return pl.kernel(
      _p15_body,
      out_type=[
          jax.ShapeDtypeStruct((NW, NBLK, BLOCK), f32),
          jax.ShapeDtypeStruct((NW, NBLK, BLOCK), f32),
          jax.ShapeDtypeStruct((NC, N_PAD), f32),
      ],
      mesh=_mesh,
      compiler_params=_sc_params,
      scratch_types=[
          pltpu.VMEM((BLOCK,), jnp.int32),
          pltpu.VMEM((BLOCK,), jnp.int32),
          pltpu.VMEM((BLOCK,), f32),
          pltpu.VMEM((BLOCK,), f32),
          pltpu.VMEM((BLOCK,), f32),
          pltpu.VMEM((BLOCK,), f32),
          pltpu.VMEM((N,), f32),
          pltpu.VMEM((N,), f32),
          pltpu.VMEM_SHARED((N_PAD,), f32),
      ],
  )(srcp, dstp, ap, flagp, dinv_n, cn_n, zeros_np)


def _p2_body(xcat, hcat, pk2, zeros_nd,
             acc_out,
             pk0_v, gidx0, hs0, xs0,
             pk1_v, gidx1, hs1, xs1,
             accum, sem0, sem1, semS0, semS1):
  c = lax.axis_index("c")
  s = lax.axis_index("s")
  bufs = ((pk0_v, gidx0, hs0, xs0, sem0, semS0),
          (pk1_v, gidx1, hs1, xs1, sem1, semS1))

  pltpu.sync_copy(zeros_nd.at[pl.ds(s * ROWS_ACC, ROWS_ACC)],
                  accum.at[pl.ds(s * ROWS_ACC, ROWS_ACC)])

  plsc.subcore_barrier()

  row_off = c * N

  def wjb(vb):
    return 2 * s + vb // NBD, vb % NBD

  def fetch_a(vb, buf):
    pk_v, gidx_v, hs_v, xs_v, sem, semS = buf

    @pl.when(vb >= 2)
    def _drain_scatter():
      pltpu.make_async_copy(xs_v, accum.at[pk_v.at[1]], semS).wait()

    wj, b = wjb(vb)
    pltpu.sync_copy(pk2.at[wj, b], pk_v)
    for k in range(BD // 16):
      sl = pl.ds(k * 16, 16)
      gidx_v[sl] = pk_v[0, sl] + row_off
    pltpu.async_copy(hcat.at[gidx_v], hs_v, sem)
    pltpu.async_copy(xcat.at[gidx_v], xs_v, sem)

  def process_a(buf):
    pk_v, gidx_v, hs_v, xs_v, sem, semS = buf
    pltpu.make_async_copy(hcat.at[gidx_v], hs_v, sem).wait()
    pltpu.make_async_copy(xcat.at[gidx_v], xs_v, sem).wait()

    def group_body(g, carry2):
      gsl = pl.ds(g * 16, 16)
      a16 = plsc.bitcast(pk_v[2, gsl], jnp.float32)
      b16 = plsc.bitcast(pk_v[3, gsl], jnp.float32)
      for e16 in range(16):
        e = g * 16 + e16
        ae = a16[e16]
        be = -b16[e16]
        for k in range(DH // 16):
          sl = pl.ds(k * 16, 16)
          xs_v[e, sl] = ae * hs_v[e, sl] + be * xs_v[e, sl]
      return carry2

    lax.fori_loop(0, BD // 16, group_body, 0)
    pltpu.async_copy(xs_v, accum.at[pk_v.at[1]], semS, add=True)

  def drain_a(buf):
    pk_v, gidx_v, hs_v, xs_v, sem, semS = buf
    pltpu.make_async_copy(xs_v, accum.at[pk_v.at[1]], semS).wait()

  def fetch_b(vb, buf):
    pk_v, gidx_v, hs_v, xs_v, sem, semS = buf

    @pl.when(vb >= 2)
    def _drain_scatter():
      pltpu.make_async_copy(hs_v, accum.at[pk_v.at[0]], semS).wait()

    wj, b = wjb(vb)
    pltpu.sync_copy(pk2.at[wj, b], pk_v)
    for k in range(BD // 16):
      sl = pl.ds(k * 16, 16)
      gidx_v[sl] = pk_v[1, sl] + row_off
    pltpu.async_copy(xcat.at[gidx_v], hs_v, sem)

  def process_b(buf):
    pk_v, gidx_v, hs_v, xs_v, sem, semS = buf
    pltpu.make_async_copy(xcat.at[gidx_v], hs_v, sem).wait()

    def group_body(g, carry2):
      b16 = plsc.bitcast(pk_v[3, pl.ds(g * 16, 16)], jnp.float32)
      for e16 in range(16):
        e = g * 16 + e16
        be = -b16[e16]
        for k in range(DH // 16):
          sl = pl.ds(k * 16, 16)
          hs_v[e, sl] = be * hs_v[e, sl]
      return carry2

    lax.fori_loop(0, BD // 16, group_body, 0)
    pltpu.async_copy(hs_v, accum.at[pk_v.at[0]], semS, add=True)

  def drain_b(buf):
    pk_v, gidx_v, hs_v, xs_v, sem, semS = buf
    pltpu.make_async_copy(hs_v, accum.at[pk_v.at[0]], semS).wait()

  NBT = 2 * NBD
  NG = NBT // 2

  def run_pass(fetch, process, drain):
    fetch(0, bufs[0])

    def pair_body(g, carry):
      fetch(2 * g + 1, bufs[1])
      process(bufs[0])

      @pl.when(g < NG - 1)
      def _prefetch():
        fetch(2 * g + 2, bufs[0])

      process(bufs[1])
      return carry

    lax.fori_loop(0, NG, pair_body, 0)
    drain(bufs[0])
    drain(bufs[1])

  run_pass(fetch_a, process_a, drain_a)
  run_pass(fetch_b, process_b, drain_b)

  plsc.subcore_barrier()

  rsl = pl.ds(s * ROWS_ACC, ROWS_ACC)
  pltpu.sync_copy(accum.at[rsl], acc_out.at[c, rsl])


def _p2(xcat, hcat, pk2, zeros_nd):
  f32 = jnp.float32
  i32 = jnp.int32
  dbuf = [
      pltpu.VMEM((4, BD), i32),
      pltpu.VMEM((BD,), i32),
      pltpu.VMEM((BD, DH), f32),
      pltpu.VMEM((BD, DH), f32),
  ]
  return pl.kernel(
      _p2_body,
      out_type=jax.ShapeDtypeStruct((NC, N_ACC, DH), f32),
      mesh=_mesh,
      compiler_params=_sc_params,
      scratch_types=dbuf + dbuf + [
          pltpu.VMEM_SHARED((N_ACC, DH), f32),
          pltpu.SemaphoreType.DMA,
          pltpu.SemaphoreType.DMA,
          pltpu.SemaphoreType.DMA,
          pltpu.SemaphoreType.DMA,
      ],
  )(xcat, hcat, pk2, zeros_nd)


def _combine_body(acc_ref, h_ref, x_ref, dsq_ref, s_ref, b_ref, o_ref):
  dsq = dsq_ref[...]
  sc = s_ref[0] + s_ref[1]
  bias = b_ref[...]
  o_ref[:, :DH] = (acc_ref[0] + dsq * h_ref[0] + sc * x_ref[:, :DH]
                   + bias[:, :DH])
  o_ref[:, DH:] = (acc_ref[1] + dsq * h_ref[1] + sc * x_ref[:, DH:]
                   + bias[:, DH:])


def _combine(acc, h3, x, dsq_col, s_part, b2d):
  RT = 400
  f32 = jnp.float32
  return pl.pallas_call(
      _combine_body,
      grid=(N // RT,),
      in_specs=[
          pl.BlockSpec((NC, RT, DH), lambda i: (0, i, 0)),
          pl.BlockSpec((NC, RT, DH), lambda i: (0, i, 0)),
          pl.BlockSpec((RT, D), lambda i: (i, 0)),
          pl.BlockSpec((RT, 1), lambda i: (i, 0)),
          pl.BlockSpec((NC, RT, 1), lambda i: (0, i, 0)),
          pl.BlockSpec((1, D), lambda i: (0, 0)),
      ],
      out_specs=pl.BlockSpec((RT, D), lambda i: (i, 0)),
      out_shape=jax.ShapeDtypeStruct((N, D), f32),
  )(acc, h3, x, dsq_col, s_part, b2d)


@jax.jit
def _run(x, edge_index, A, weight, temperature, norm_energies, W, b):
  f32 = jnp.float32
  i32 = jnp.int32
  src = edge_index[0]
  dst = edge_index[1]

  pad = E_PAD - E
  shp128 = (NW, NBLK, BLOCK)
  shp64 = (NW, NBD, BD)
  src_f = jnp.concatenate([src, jnp.zeros((pad,), i32)])
  dst_f = jnp.concatenate([dst, jnp.zeros((pad,), i32)])
  a_f = jnp.concatenate([A, jnp.zeros((pad,), f32)])
  flag_f = jnp.concatenate([jnp.ones((E,), f32), jnp.zeros((pad,), f32)])
  zeros_np = jnp.zeros((N_PAD,), f32)
  zeros_nd = jnp.zeros((N_ACC, DH), f32)

  h3 = _matmul(x, W)
  hcat = h3.reshape(2 * N, DH)
  xcat = x.reshape(N, 2, DH).transpose(1, 0, 2).reshape(2 * N, DH)

  pk1 = jnp.stack(
      [src_f.reshape(shp64), dst_f.reshape(shp64),
       lax.bitcast_convert_type(a_f, i32).reshape(shp64),
       lax.bitcast_convert_type(flag_f, i32).reshape(shp64)], axis=2)

  en_part, deg_part = _p1(x, pk1, zeros_np)
  dinv2d, cn2d, dsq2d = _node_math(en_part, deg_part, temperature, weight,
                                   norm_energies)
  dinv_n = dinv2d.reshape(-1)[:N]
  cn_n = cn2d.reshape(-1)[:N]

  alphap, svalp, s_part = _p15(src_f.reshape(shp128), dst_f.reshape(shp128),
                               a_f.reshape(shp128), flag_f.reshape(shp128),
                               dinv_n, cn_n, zeros_np)
  pk2 = jnp.stack(
      [src_f.reshape(shp64), dst_f.reshape(shp64),
       lax.bitcast_convert_type(alphap, i32).reshape(shp64),
       lax.bitcast_convert_type(svalp, i32).reshape(shp64)], axis=2)
  acc = _p2(xcat, hcat, pk2, zeros_nd)

  out = _combine(acc, h3, x, dsq2d.reshape(-1)[:N].reshape(N, 1),
                 s_part[:, :N].reshape(NC, N, 1), b.reshape(1, D))
  return out


def kernel(x, edge_index, A, weight, temperature, norm_energies, W, b):
  return _run(x, edge_index, A, weight, temperature, norm_energies, W, b)


# speedup vs baseline: 4.1886x; 1.0026x over previous
import jax
import jax.numpy as jnp
from jax import lax
from jax.experimental import pallas as pl
from jax.experimental.pallas import tpu as pltpu
from jax.experimental.pallas import tpu_sc as plsc

N = 10000
E = 160000
D = 256
DH = 128
EPS = 1e-12

NC = 2
NS = 16
NW = NC * NS

BLOCK = 128
EPT_PAD = 5120
NBLK = EPT_PAD // BLOCK
BD = 64
NBD = EPT_PAD // BD
E_PAD = EPT_PAD * NW
N_PAD = 10240
N_ACC = 10112
ROWS_ACC = N_ACC // NS

_mesh = plsc.VectorSubcoreMesh(
    core_axis_name="c", subcore_axis_name="s", num_cores=NC, num_subcores=NS)
_sc_params = pltpu.CompilerParams(needs_layout_passes=False)


def _matmul_body(x_ref, w_ref, o_ref):
  o_ref[0] = jnp.dot(x_ref[...], w_ref[...],
                     preferred_element_type=jnp.float32)


def _matmul(x, W):
  RT = 400
  return pl.pallas_call(
      _matmul_body,
      grid=(N // RT, 2),
      in_specs=[
          pl.BlockSpec((RT, D), lambda i, c: (i, 0)),
          pl.BlockSpec((D, DH), lambda i, c: (0, c)),
      ],
      out_specs=pl.BlockSpec((1, RT, DH), lambda i, c: (c, i, 0)),
      out_shape=jax.ShapeDtypeStruct((2, N, DH), jnp.float32),
  )(x, W)


def _p1_body(x_hbm, pk1, zeros_np,
             en_out, deg_out,
             pk0_v, xs0, xd0, eb0, fb0,
             pk1_v, xs1, xd1, eb1, fb1,
             tbuf, en_acc, deg_acc, sem0, sem1):
  c = lax.axis_index("c")
  s = lax.axis_index("s")
  w = c * NS + s
  bufs = ((pk0_v, xs0, xd0, eb0, fb0, sem0),
          (pk1_v, xs1, xd1, eb1, fb1, sem1))

  @pl.when(s == 0)
  def _init():
    pltpu.sync_copy(zeros_np, en_acc)
    pltpu.sync_copy(zeros_np, deg_acc)

  plsc.subcore_barrier()

  def fetch(b, buf):
    pk_v, xs_v, xd_v, _, _, sem = buf
    pltpu.sync_copy(pk1.at[w, b], pk_v)
    pltpu.async_copy(x_hbm.at[pk_v.at[0]], xs_v, sem)
    pltpu.async_copy(x_hbm.at[pk_v.at[1]], xd_v, sem)

  def process(buf):
    pk_v, xs_v, xd_v, e_buf, f_buf, sem = buf
    pltpu.make_async_copy(x_hbm.at[pk_v.at[0]], xs_v, sem).wait()
    pltpu.make_async_copy(x_hbm.at[pk_v.at[1]], xd_v, sem).wait()
    lane17 = lax.iota(jnp.int32, 16) * 17

    def group_body(g, carry2):
      sl = pl.ds(g * 16, 16)
      a16 = plsc.bitcast(pk_v[2, sl], jnp.float32)
      f_buf[sl] = plsc.bitcast(pk_v[3, sl], jnp.float32)
      for e16 in range(16):
        e = g * 16 + e16
        acc = jnp.zeros((16,), jnp.float32)
        for k in range(D // 32):
          sp = plsc.bitcast(xs_v[e, pl.ds(k * 16, 16)], jnp.bfloat16)
          dp = plsc.bitcast(xd_v[e, pl.ds(k * 16, 16)], jnp.bfloat16)
          sa, sb = plsc.unpack(sp, format=plsc.PackFormat.INTERLEAVED)
          da, db = plsc.unpack(dp, format=plsc.PackFormat.INTERLEAVED)
          d0 = sa - da
          d1 = sb - db
          acc = acc + d0 * d0 + d1 * d1
        tbuf[pl.ds(e16 * 17, 16)] = acc
      esum = jnp.zeros((16,), jnp.float32)
      for ccol in range(16):
        esum = esum + plsc.load_gather(tbuf, [lane17 + ccol])
      e_buf[sl] = a16 * esum
      return carry2

    lax.fori_loop(0, BD // 16, group_body, 0)
    pltpu.sync_copy(e_buf, en_acc.at[pk_v.at[1]], add=True)
    pltpu.sync_copy(f_buf, deg_acc.at[pk_v.at[1]], add=True)

  fetch(0, bufs[0])
  NG = NBD // 2

  def pair_body(g, carry):
    fetch(2 * g + 1, bufs[1])
    process(bufs[0])

    @pl.when(g < NG - 1)
    def _prefetch():
      fetch(2 * g + 2, bufs[0])

    process(bufs[1])
    return carry

  lax.fori_loop(0, NG, pair_body, 0)

  plsc.subcore_barrier()

  @pl.when(s == 0)
  def _writeout():
    pltpu.sync_copy(en_acc, en_out.at[c])
    pltpu.sync_copy(deg_acc, deg_out.at[c])


def _p1(x, pk1, zeros_np):
  f32 = jnp.float32
  i32 = jnp.int32
  dbuf = [
      pltpu.VMEM((4, BD), i32),
      pltpu.VMEM((BD, D // 2), i32),
      pltpu.VMEM((BD, D // 2), i32),
      pltpu.VMEM((BD,), f32),
      pltpu.VMEM((BD,), f32),
  ]
  return pl.kernel(
      _p1_body,
      out_type=[
          jax.ShapeDtypeStruct((NC, N_PAD), f32),
          jax.ShapeDtypeStruct((NC, N_PAD), f32),
      ],
      mesh=_mesh,
      compiler_params=_sc_params,
      scratch_types=dbuf + dbuf + [
          pltpu.VMEM((16 * 17,), f32),
          pltpu.VMEM_SHARED((N_PAD,), f32),
          pltpu.VMEM_SHARED((N_PAD,), f32),
          pltpu.SemaphoreType.DMA,
          pltpu.SemaphoreType.DMA,
      ],
  )(x, pk1, zeros_np)


def _node_body(ep_ref, dp_ref, t_ref, w_ref, ne_ref, dinv_ref, cn_ref,
               dsq_ref):
  en = ep_ref[0] + ep_ref[1]
  deg = dp_ref[0] + dp_ref[1] + 1.0
  row = lax.broadcasted_iota(jnp.int32, (N_PAD // 128, 128), 0)
  col = lax.broadcasted_iota(jnp.int32, (N_PAD // 128, 128), 1)
  valid = (row * 128 + col) < N

  T = t_ref[0, 0]
  wgt = w_ref[0, 0]
  do_norm = ne_ref[0, 0] != 0
  neg_inf = jnp.float32(-jnp.inf)

  m = jnp.max(jnp.where(valid, en, neg_inf))
  r = 1.0 / (m + EPS)
  en_n = jnp.where(do_norm, en * r, en)

  z = -en_n / T
  zmax = jnp.max(jnp.where(valid, z, neg_inf))
  ez = jnp.where(valid, jnp.exp(z - zmax), 0.0)
  p = ez / jnp.sum(ez)
  g = -(jnp.log(p + EPS) + p / (p + EPS))
  pg = jnp.sum(jnp.where(valid, p * g, 0.0))
  u = (-1.0 / T) * p * (g - pg)
  sum_ue = jnp.sum(jnp.where(valid, u * en, 0.0))
  is_max = jnp.where(valid & (en == m), 1.0, 0.0)
  ties = jnp.sum(is_max)
  v = jnp.where(do_norm, u * r - (r * r) * sum_ue * is_max / ties, u)

  dinv = lax.rsqrt(deg)
  dinv_ref[...] = dinv
  cn_ref[...] = 2.0 * wgt * v
  dsq_ref[...] = 1.0 / deg


def _node_math(en_part, deg_part, temperature, weight, norm_energies):
  f32 = jnp.float32
  shp = (N_PAD // 128, 128)
  return pl.pallas_call(
      _node_body,
      in_specs=[
          pl.BlockSpec((NC,) + shp, lambda: (0, 0, 0)),
          pl.BlockSpec((NC,) + shp, lambda: (0, 0, 0)),
          pl.BlockSpec((1, 1), lambda: (0, 0)),
          pl.BlockSpec((1, 1), lambda: (0, 0)),
          pl.BlockSpec((1, 1), lambda: (0, 0)),
      ],
      out_specs=[
          pl.BlockSpec(shp, lambda: (0, 0)),
          pl.BlockSpec(shp, lambda: (0, 0)),
          pl.BlockSpec(shp, lambda: (0, 0)),
      ],
      out_shape=[
          jax.ShapeDtypeStruct(shp, f32),
          jax.ShapeDtypeStruct(shp, f32),
          jax.ShapeDtypeStruct(shp, f32),
      ],
  )(en_part.reshape((NC,) + shp), deg_part.reshape((NC,) + shp),
    temperature.reshape(1, 1), weight.reshape(1, 1),
    jnp.asarray(norm_energies, jnp.int32).reshape(1, 1))


def _p15_body(srcp, dstp, ap, flagp, dinv_hbm, cn_hbm, zeros_np,
              alpha_out, sval_out, s_out,
              sidx_v, didx_v, a_v, f_v, al_v, sv_v, dinv_t, cn_t,
              sacc):
  c = lax.axis_index("c")
  s = lax.axis_index("s")
  w = c * NS + s

  pltpu.sync_copy(dinv_hbm, dinv_t)
  pltpu.sync_copy(cn_hbm, cn_t)

  @pl.when(s == 0)
  def _init():
    pltpu.sync_copy(zeros_np, sacc)

  plsc.subcore_barrier()

  def block_body(b, carry):
    pltpu.sync_copy(srcp.at[w, b], sidx_v)
    pltpu.sync_copy(dstp.at[w, b], didx_v)
    pltpu.sync_copy(ap.at[w, b], a_v)
    pltpu.sync_copy(flagp.at[w, b], f_v)
    for k in range(BLOCK // 16):
      sl = pl.ds(k * 16, 16)
      idx_s = sidx_v[sl]
      idx_d = didx_v[sl]
      dv_s = plsc.load_gather(dinv_t, [idx_s])
      dv_d = plsc.load_gather(dinv_t, [idx_d])
      cn_d = plsc.load_gather(cn_t, [idx_d])
      al_v[sl] = f_v[sl] * dv_s * dv_d
      sv_v[sl] = a_v[sl] * cn_d
    pltpu.sync_copy(al_v, alpha_out.at[w, b])
    pltpu.sync_copy(sv_v, sval_out.at[w, b])
    pltpu.sync_copy(sv_v, sacc.at[sidx_v], add=True)
    pltpu.sync_copy(sv_v, sacc.at[didx_v], add=True)
    return carry

  lax.fori_loop(0, NBLK, block_body, 0)

  plsc.subcore_barrier()

  @pl.when(s == 0)
  def _writeout():
    pltpu.sync_copy(sacc, s_out.at[c])


def _p15(srcp, dstp, ap, flagp, dinv_n, cn_n, zeros_np):
  f32 = jnp.float32
  return pl.kernel(
      _p15_body,
      out_type=[
          jax.ShapeDtypeStruct((NW, NBLK, BLOCK), f32),
          jax.ShapeDtypeStruct((NW, NBLK, BLOCK), f32),
          jax.ShapeDtypeStruct((NC, N_PAD), f32),
      ],
      mesh=_mesh,
      compiler_params=_sc_params,
      scratch_types=[
          pltpu.VMEM((BLOCK,), jnp.int32),
          pltpu.VMEM((BLOCK,), jnp.int32),
          pltpu.VMEM((BLOCK,), f32),
          pltpu.VMEM((BLOCK,), f32),
          pltpu.VMEM((BLOCK,), f32),
          pltpu.VMEM((BLOCK,), f32),
          pltpu.VMEM((N,), f32),
          pltpu.VMEM((N,), f32),
          pltpu.VMEM_SHARED((N_PAD,), f32),
      ],
  )(srcp, dstp, ap, flagp, dinv_n, cn_n, zeros_np)


def _p2_body(xcat, hcat, pk2, zeros_nd,
             acc_out,
             pk0_v, gidx0, hs0, xs0,
             pk1_v, gidx1, hs1, xs1,
             accum, sem0, sem1, semS0, semS1):
  c = lax.axis_index("c")
  s = lax.axis_index("s")
  bufs = ((pk0_v, gidx0, hs0, xs0, sem0, semS0),
          (pk1_v, gidx1, hs1, xs1, sem1, semS1))

  pltpu.sync_copy(zeros_nd.at[pl.ds(s * ROWS_ACC, ROWS_ACC)],
                  accum.at[pl.ds(s * ROWS_ACC, ROWS_ACC)])

  plsc.subcore_barrier()

  row_off = c * N

  def wjb(vb):
    return 2 * s + vb // NBD, vb % NBD

  def fetch_a(vb, buf):
    pk_v, gidx_v, hs_v, xs_v, sem, semS = buf

    @pl.when(vb >= 2)
    def _drain_scatter():
      pltpu.make_async_copy(xs_v, accum.at[pk_v.at[1]], semS).wait()

    wj, b = wjb(vb)
    pltpu.sync_copy(pk2.at[wj, b], pk_v)
    for k in range(BD // 16):
      sl = pl.ds(k * 16, 16)
      gidx_v[sl] = pk_v[0, sl] + row_off
    pltpu.async_copy(hcat.at[gidx_v], hs_v, sem)
    pltpu.async_copy(xcat.at[gidx_v], xs_v, sem)

  def process_a(buf):
    pk_v, gidx_v, hs_v, xs_v, sem, semS = buf
    pltpu.make_async_copy(hcat.at[gidx_v], hs_v, sem).wait()
    pltpu.make_async_copy(xcat.at[gidx_v], xs_v, sem).wait()

    def group_body(g, carry2):
      gsl = pl.ds(g * 16, 16)
      a16 = plsc.bitcast(pk_v[2, gsl], jnp.float32)
      b16 = plsc.bitcast(pk_v[3, gsl], jnp.float32)
      for e16 in range(16):
        e = g * 16 + e16
        ae = a16[e16]
        be = -b16[e16]
        for k in range(DH // 16):
          sl = pl.ds(k * 16, 16)
          xs_v[e, sl] = ae * hs_v[e, sl] + be * xs_v[e, sl]
      return carry2

    lax.fori_loop(0, BD // 16, group_body, 0)
    pltpu.async_copy(xs_v, accum.at[pk_v.at[1]], semS, add=True)

  def drain_a(buf):
    pk_v, gidx_v, hs_v, xs_v, sem, semS = buf
    pltpu.make_async_copy(xs_v, accum.at[pk_v.at[1]], semS).wait()

  def fetch_b(vb, buf):
    pk_v, gidx_v, hs_v, xs_v, sem, semS = buf

    @pl.when(vb >= 2)
    def _drain_scatter():
      pltpu.make_async_copy(hs_v, accum.at[pk_v.at[0]], semS).wait()

    wj, b = wjb(vb)
    pltpu.sync_copy(pk2.at[wj, b], pk_v)
    for k in range(BD // 16):
      sl = pl.ds(k * 16, 16)
      gidx_v[sl] = pk_v[1, sl] + row_off
    pltpu.async_copy(xcat.at[gidx_v], hs_v, sem)

  def process_b(buf):
    pk_v, gidx_v, hs_v, xs_v, sem, semS = buf
    pltpu.make_async_copy(xcat.at[gidx_v], hs_v, sem).wait()

    def group_body(g, carry2):
      b16 = plsc.bitcast(pk_v[3, pl.ds(g * 16, 16)], jnp.float32)
      for e16 in range(16):
        e = g * 16 + e16
        be = -b16[e16]
        for k in range(DH // 16):
          sl = pl.ds(k * 16, 16)
          hs_v[e, sl] = be * hs_v[e, sl]
      return carry2

    lax.fori_loop(0, BD // 16, group_body, 0)
    pltpu.async_copy(hs_v, accum.at[pk_v.at[0]], semS, add=True)

  def drain_b(buf):
    pk_v, gidx_v, hs_v, xs_v, sem, semS = buf
    pltpu.make_async_copy(hs_v, accum.at[pk_v.at[0]], semS).wait()

  NBT = 2 * NBD
  NG = NBT // 2

  def run_pass(fetch, process, drain):
    fetch(0, bufs[0])

    def pair_body(g, carry):
      fetch(2 * g + 1, bufs[1])
      process(bufs[0])

      @pl.when(g < NG - 1)
      def _prefetch():
        fetch(2 * g + 2, bufs[0])

      process(bufs[1])
      return carry

    lax.fori_loop(0, NG, pair_body, 0)
    drain(bufs[0])
    drain(bufs[1])

  run_pass(fetch_a, process_a, drain_a)
  run_pass(fetch_b, process_b, drain_b)

  plsc.subcore_barrier()

  rsl = pl.ds(s * ROWS_ACC, ROWS_ACC)
  pltpu.sync_copy(accum.at[rsl], acc_out.at[c, rsl])


def _p2(xcat, hcat, pk2, zeros_nd):
  f32 = jnp.float32
  i32 = jnp.int32
  dbuf = [
      pltpu.VMEM((4, BD), i32),
      pltpu.VMEM((BD,), i32),
      pltpu.VMEM((BD, DH), f32),
      pltpu.VMEM((BD, DH), f32),
  ]
  return pl.kernel(
      _p2_body,
      out_type=jax.ShapeDtypeStruct((NC, N_ACC, DH), f32),
      mesh=_mesh,
      compiler_params=_sc_params,
      scratch_types=dbuf + dbuf + [
          pltpu.VMEM_SHARED((N_ACC, DH), f32),
          pltpu.SemaphoreType.DMA,
          pltpu.SemaphoreType.DMA,
          pltpu.SemaphoreType.DMA,
          pltpu.SemaphoreType.DMA,
      ],
  )(xcat, hcat, pk2, zeros_nd)


def _combine_body(acc_ref, h_ref, x_ref, dsq_ref, s_ref, b_ref, o_ref):
  dsq = dsq_ref[...]
  sc = s_ref[0] + s_ref[1]
  bias = b_ref[...]
  o_ref[:, :DH] = (acc_ref[0] + dsq * h_ref[0] + sc * x_ref[:, :DH]
                   + bias[:, :DH])
  o_ref[:, DH:] = (acc_ref[1] + dsq * h_ref[1] + sc * x_ref[:, DH:]
                   + bias[:, DH:])


def _combine(acc, h3, x, dsq_col, s_part, b2d):
  RT = 400
  f32 = jnp.float32
  return pl.pallas_call(
      _combine_body,
      grid=(N // RT,),
      in_specs=[
          pl.BlockSpec((NC, RT, DH), lambda i: (0, i, 0)),
          pl.BlockSpec((NC, RT, DH), lambda i: (0, i, 0)),
          pl.BlockSpec((RT, D), lambda i: (i, 0)),
          pl.BlockSpec((RT, 1), lambda i: (i, 0)),
          pl.BlockSpec((NC, RT, 1), lambda i: (0, i, 0)),
          pl.BlockSpec((1, D), lambda i: (0, 0)),
      ],
      out_specs=pl.BlockSpec((RT, D), lambda i: (i, 0)),
      out_shape=jax.ShapeDtypeStruct((N, D), f32),
  )(acc, h3, x, dsq_col, s_part, b2d)


@jax.jit
def _run(x, edge_index, A, weight, temperature, norm_energies, W, b):
  f32 = jnp.float32
  i32 = jnp.int32
  src = edge_index[0]
  dst = edge_index[1]

  pad = E_PAD - E
  shp128 = (NW, NBLK, BLOCK)
  shp64 = (NW, NBD, BD)
  src_f = jnp.concatenate([src, jnp.zeros((pad,), i32)])
  dst_f = jnp.concatenate([dst, jnp.zeros((pad,), i32)])
  a_f = jnp.concatenate([A, jnp.zeros((pad,), f32)])
  flag_f = jnp.concatenate([jnp.ones((E,), f32), jnp.zeros((pad,), f32)])
  zeros_np = jnp.zeros((N_PAD,), f32)
  zeros_nd = jnp.zeros((N_ACC, DH), f32)

  h3 = _matmul(x, W)
  hcat = h3.reshape(2 * N, DH)
  xcat = x.reshape(N, 2, DH).transpose(1, 0, 2).reshape(2 * N, DH)

  pk1 = jnp.stack(
      [src_f.reshape(shp64), dst_f.reshape(shp64),
       lax.bitcast_convert_type(a_f, i32).reshape(shp64),
       lax.bitcast_convert_type(flag_f, i32).reshape(shp64)], axis=2)

  xbi = lax.bitcast_convert_type(
      x.astype(jnp.bfloat16).reshape(N, D // 2, 2), i32).reshape(N, D // 2)
  en_part, deg_part = _p1(xbi, pk1, zeros_np)
  dinv2d, cn2d, dsq2d = _node_math(en_part, deg_part, temperature, weight,
                                   norm_energies)
  dinv_n = dinv2d.reshape(-1)[:N]
  cn_n = cn2d.reshape(-1)[:N]

  alphap, svalp, s_part = _p15(src_f.reshape(shp128), dst_f.reshape(shp128),
                               a_f.reshape(shp128), flag_f.reshape(shp128),
                               dinv_n, cn_n, zeros_np)
  pk2 = jnp.stack(
      [src_f.reshape(shp64), dst_f.reshape(shp64),
       lax.bitcast_convert_type(alphap, i32).reshape(shp64),
       lax.bitcast_convert_type(svalp, i32).reshape(shp64)], axis=2)
  acc = _p2(xcat, hcat, pk2, zeros_nd)

  out = _combine(acc, h3, x, dsq2d.reshape(-1)[:N].reshape(N, 1),
                 s_part[:, :N].reshape(NC, N, 1), b.reshape(1, D))
  return out


def kernel(x, edge_index, A, weight, temperature, norm_energies, W, b):
  return _run(x, edge_index, A, weight, temperature, norm_energies, W, b)


# speedup vs baseline: 4.5307x; 1.0817x over previous
import jax
import jax.numpy as jnp
from jax import lax
from jax.experimental import pallas as pl
from jax.experimental.pallas import tpu as pltpu
from jax.experimental.pallas import tpu_sc as plsc

N = 10000
E = 160000
D = 256
DH = 128
EPS = 1e-12

NC = 2
NS = 16
NW = NC * NS

BLOCK = 128
EPT_PAD = 5120
NBLK = EPT_PAD // BLOCK
BD = 64
NBD = EPT_PAD // BD
BP1 = 128
NB1 = EPT_PAD // BP1
E_PAD = EPT_PAD * NW
N_PAD = 10240
N_ACC = 10112
ROWS_ACC = N_ACC // NS

_mesh = plsc.VectorSubcoreMesh(
    core_axis_name="c", subcore_axis_name="s", num_cores=NC, num_subcores=NS)
_sc_params = pltpu.CompilerParams(needs_layout_passes=False)


def _matmul_body(x_ref, w_ref, o_ref):
  o_ref[0] = jnp.dot(x_ref[...], w_ref[...],
                     preferred_element_type=jnp.float32)


def _matmul(x, W):
  RT = 400
  return pl.pallas_call(
      _matmul_body,
      grid=(N // RT, 2),
      in_specs=[
          pl.BlockSpec((RT, D), lambda i, c: (i, 0)),
          pl.BlockSpec((D, DH), lambda i, c: (0, c)),
      ],
      out_specs=pl.BlockSpec((1, RT, DH), lambda i, c: (c, i, 0)),
      out_shape=jax.ShapeDtypeStruct((2, N, DH), jnp.float32),
  )(x, W)


def _p1_body(x_hbm, pk1, zeros_np,
             en_out, deg_out,
             pk0_v, xs0, xd0, eb0, fb0,
             pk1_v, xs1, xd1, eb1, fb1,
             tbuf, en_acc, deg_acc, sem0, sem1):
  c = lax.axis_index("c")
  s = lax.axis_index("s")
  w = c * NS + s
  bufs = ((pk0_v, xs0, xd0, eb0, fb0, sem0),
          (pk1_v, xs1, xd1, eb1, fb1, sem1))

  @pl.when(s == 0)
  def _init():
    pltpu.sync_copy(zeros_np, en_acc)
    pltpu.sync_copy(zeros_np, deg_acc)

  plsc.subcore_barrier()

  def fetch(b, buf):
    pk_v, xs_v, xd_v, _, _, sem = buf
    pltpu.sync_copy(pk1.at[w, b], pk_v)
    pltpu.async_copy(x_hbm.at[pk_v.at[0]], xs_v, sem)
    pltpu.async_copy(x_hbm.at[pk_v.at[1]], xd_v, sem)

  def process(buf):
    pk_v, xs_v, xd_v, e_buf, f_buf, sem = buf
    pltpu.make_async_copy(x_hbm.at[pk_v.at[0]], xs_v, sem).wait()
    pltpu.make_async_copy(x_hbm.at[pk_v.at[1]], xd_v, sem).wait()
    lane17 = lax.iota(jnp.int32, 16) * 17

    def group_body(g, carry2):
      sl = pl.ds(g * 16, 16)
      a16 = plsc.bitcast(pk_v[2, sl], jnp.float32)
      f_buf[sl] = plsc.bitcast(pk_v[3, sl], jnp.float32)
      for e16 in range(16):
        e = g * 16 + e16
        acc = jnp.zeros((16,), jnp.float32)
        for k in range(D // 32):
          sp = plsc.bitcast(xs_v[e, pl.ds(k * 16, 16)], jnp.bfloat16)
          dp = plsc.bitcast(xd_v[e, pl.ds(k * 16, 16)], jnp.bfloat16)
          sa, sb = plsc.unpack(sp, format=plsc.PackFormat.INTERLEAVED)
          da, db = plsc.unpack(dp, format=plsc.PackFormat.INTERLEAVED)
          d0 = sa - da
          d1 = sb - db
          acc = acc + d0 * d0 + d1 * d1
        tbuf[pl.ds(e16 * 17, 16)] = acc
      esum = jnp.zeros((16,), jnp.float32)
      for ccol in range(16):
        esum = esum + plsc.load_gather(tbuf, [lane17 + ccol])
      e_buf[sl] = a16 * esum
      return carry2

    lax.fori_loop(0, BP1 // 16, group_body, 0)
    pltpu.sync_copy(e_buf, en_acc.at[pk_v.at[1]], add=True)
    pltpu.sync_copy(f_buf, deg_acc.at[pk_v.at[1]], add=True)

  fetch(0, bufs[0])
  NG = NB1 // 2

  def pair_body(g, carry):
    fetch(2 * g + 1, bufs[1])
    process(bufs[0])

    @pl.when(g < NG - 1)
    def _prefetch():
      fetch(2 * g + 2, bufs[0])

    process(bufs[1])
    return carry

  lax.fori_loop(0, NG, pair_body, 0)

  plsc.subcore_barrier()

  @pl.when(s == 0)
  def _writeout():
    pltpu.sync_copy(en_acc, en_out.at[c])
    pltpu.sync_copy(deg_acc, deg_out.at[c])


def _p1(x, pk1, zeros_np):
  f32 = jnp.float32
  i32 = jnp.int32
  dbuf = [
      pltpu.VMEM((4, BP1), i32),
      pltpu.VMEM((BP1, D // 2), i32),
      pltpu.VMEM((BP1, D // 2), i32),
      pltpu.VMEM((BP1,), f32),
      pltpu.VMEM((BP1,), f32),
  ]
  return pl.kernel(
      _p1_body,
      out_type=[
          jax.ShapeDtypeStruct((NC, N_PAD), f32),
          jax.ShapeDtypeStruct((NC, N_PAD), f32),
      ],
      mesh=_mesh,
      compiler_params=_sc_params,
      scratch_types=dbuf + dbuf + [
          pltpu.VMEM((16 * 17,), f32),
          pltpu.VMEM_SHARED((N_PAD,), f32),
          pltpu.VMEM_SHARED((N_PAD,), f32),
          pltpu.SemaphoreType.DMA,
          pltpu.SemaphoreType.DMA,
      ],
  )(x, pk1, zeros_np)


def _node_body(ep_ref, dp_ref, t_ref, w_ref, ne_ref, dinv_ref, cn_ref,
               dsq_ref):
  en = ep_ref[0] + ep_ref[1]
  deg = dp_ref[0] + dp_ref[1] + 1.0
  row = lax.broadcasted_iota(jnp.int32, (N_PAD // 128, 128), 0)
  col = lax.broadcasted_iota(jnp.int32, (N_PAD // 128, 128), 1)
  valid = (row * 128 + col) < N

  T = t_ref[0, 0]
  wgt = w_ref[0, 0]
  do_norm = ne_ref[0, 0] != 0
  neg_inf = jnp.float32(-jnp.inf)

  m = jnp.max(jnp.where(valid, en, neg_inf))
  r = 1.0 / (m + EPS)
  en_n = jnp.where(do_norm, en * r, en)

  z = -en_n / T
  zmax = jnp.max(jnp.where(valid, z, neg_inf))
  ez = jnp.where(valid, jnp.exp(z - zmax), 0.0)
  p = ez / jnp.sum(ez)
  g = -(jnp.log(p + EPS) + p / (p + EPS))
  pg = jnp.sum(jnp.where(valid, p * g, 0.0))
  u = (-1.0 / T) * p * (g - pg)
  sum_ue = jnp.sum(jnp.where(valid, u * en, 0.0))
  is_max = jnp.where(valid & (en == m), 1.0, 0.0)
  ties = jnp.sum(is_max)
  v = jnp.where(do_norm, u * r - (r * r) * sum_ue * is_max / ties, u)

  dinv = lax.rsqrt(deg)
  dinv_ref[...] = dinv
  cn_ref[...] = 2.0 * wgt * v
  dsq_ref[...] = 1.0 / deg


def _node_math(en_part, deg_part, temperature, weight, norm_energies):
  f32 = jnp.float32
  shp = (N_PAD // 128, 128)
  return pl.pallas_call(
      _node_body,
      in_specs=[
          pl.BlockSpec((NC,) + shp, lambda: (0, 0, 0)),
          pl.BlockSpec((NC,) + shp, lambda: (0, 0, 0)),
          pl.BlockSpec((1, 1), lambda: (0, 0)),
          pl.BlockSpec((1, 1), lambda: (0, 0)),
          pl.BlockSpec((1, 1), lambda: (0, 0)),
      ],
      out_specs=[
          pl.BlockSpec(shp, lambda: (0, 0)),
          pl.BlockSpec(shp, lambda: (0, 0)),
          pl.BlockSpec(shp, lambda: (0, 0)),
      ],
      out_shape=[
          jax.ShapeDtypeStruct(shp, f32),
          jax.ShapeDtypeStruct(shp, f32),
          jax.ShapeDtypeStruct(shp, f32),
      ],
  )(en_part.reshape((NC,) + shp), deg_part.reshape((NC,) + shp),
    temperature.reshape(1, 1), weight.reshape(1, 1),
    jnp.asarray(norm_energies, jnp.int32).reshape(1, 1))


def _p15_body(srcp, dstp, ap, flagp, dinv_hbm, cn_hbm, zeros_np,
              alpha_out, sval_out, s_out,
              sidx_v, didx_v, a_v, f_v, al_v, sv_v, dinv_t, cn_t,
              sacc):
  c = lax.axis_index("c")
  s = lax.axis_index("s")
  w = c * NS + s

  pltpu.sync_copy(dinv_hbm, dinv_t)
  pltpu.sync_copy(cn_hbm, cn_t)

  @pl.when(s == 0)
  def _init():
    pltpu.sync_copy(zeros_np, sacc)

  plsc.subcore_barrier()

  def block_body(b, carry):
    pltpu.sync_copy(srcp.at[w, b], sidx_v)
    pltpu.sync_copy(dstp.at[w, b], didx_v)
    pltpu.sync_copy(ap.at[w, b], a_v)
    pltpu.sync_copy(flagp.at[w, b], f_v)
    for k in range(BLOCK // 16):
      sl = pl.ds(k * 16, 16)
      idx_s = sidx_v[sl]
      idx_d = didx_v[sl]
      dv_s = plsc.load_gather(dinv_t, [idx_s])
      dv_d = plsc.load_gather(dinv_t, [idx_d])
      cn_d = plsc.load_gather(cn_t, [idx_d])
      al_v[sl] = f_v[sl] * dv_s * dv_d
      sv_v[sl] = a_v[sl] * cn_d
    pltpu.sync_copy(al_v, alpha_out.at[w, b])
    pltpu.sync_copy(sv_v, sval_out.at[w, b])
    pltpu.sync_copy(sv_v, sacc.at[sidx_v], add=True)
    pltpu.sync_copy(sv_v, sacc.at[didx_v], add=True)
    return carry

  lax.fori_loop(0, NBLK, block_body, 0)

  plsc.subcore_barrier()

  @pl.when(s == 0)
  def _writeout():
    pltpu.sync_copy(sacc, s_out.at[c])


def _p15(srcp, dstp, ap, flagp, dinv_n, cn_n, zeros_np):
  f32 = jnp.float32
  return pl.kernel(
      _p15_body,
      out_type=[
          jax.ShapeDtypeStruct((NW, NBLK, BLOCK), f32),
          jax.ShapeDtypeStruct((NW, NBLK, BLOCK), f32),
          jax.ShapeDtypeStruct((NC, N_PAD), f32),
      ],
      mesh=_mesh,
      compiler_params=_sc_params,
      scratch_types=[
          pltpu.VMEM((BLOCK,), jnp.int32),
          pltpu.VMEM((BLOCK,), jnp.int32),
          pltpu.VMEM((BLOCK,), f32),
          pltpu.VMEM((BLOCK,), f32),
          pltpu.VMEM((BLOCK,), f32),
          pltpu.VMEM((BLOCK,), f32),
          pltpu.VMEM((N,), f32),
          pltpu.VMEM((N,), f32),
          pltpu.VMEM_SHARED((N_PAD,), f32),
      ],
  )(srcp, dstp, ap, flagp, dinv_n, cn_n, zeros_np)


def _p2_body(xcat, hcat, pk2, zeros_nd,
             acc_out,
             pk0_v, gidx0, hs0, xs0,
             pk1_v, gidx1, hs1, xs1,
             accum, sem0, sem1, semS0, semS1):
  c = lax.axis_index("c")
  s = lax.axis_index("s")
  bufs = ((pk0_v, gidx0, hs0, xs0, sem0, semS0),
          (pk1_v, gidx1, hs1, xs1, sem1, semS1))

  pltpu.sync_copy(zeros_nd.at[pl.ds(s * ROWS_ACC, ROWS_ACC)],
                  accum.at[pl.ds(s * ROWS_ACC, ROWS_ACC)])

  plsc.subcore_barrier()

  row_off = c * N

  def wjb(vb):
    return 2 * s + vb // NBD, vb % NBD

  def fetch_a(vb, buf):
    pk_v, gidx_v, hs_v, xs_v, sem, semS = buf

    @pl.when(vb >= 2)
    def _drain_scatter():
      pltpu.make_async_copy(xs_v, accum.at[pk_v.at[1]], semS).wait()

    wj, b = wjb(vb)
    pltpu.sync_copy(pk2.at[wj, b], pk_v)
    for k in range(BD // 16):
      sl = pl.ds(k * 16, 16)
      gidx_v[sl] = pk_v[0, sl] + row_off
    pltpu.async_copy(hcat.at[gidx_v], hs_v, sem)
    pltpu.async_copy(xcat.at[gidx_v], xs_v, sem)

  def process_a(buf):
    pk_v, gidx_v, hs_v, xs_v, sem, semS = buf
    pltpu.make_async_copy(hcat.at[gidx_v], hs_v, sem).wait()
    pltpu.make_async_copy(xcat.at[gidx_v], xs_v, sem).wait()

    def group_body(g, carry2):
      gsl = pl.ds(g * 16, 16)
      a16 = plsc.bitcast(pk_v[2, gsl], jnp.float32)
      b16 = plsc.bitcast(pk_v[3, gsl], jnp.float32)
      for e16 in range(16):
        e = g * 16 + e16
        ae = a16[e16]
        be = -b16[e16]
        for k in range(DH // 16):
          sl = pl.ds(k * 16, 16)
          xs_v[e, sl] = ae * hs_v[e, sl] + be * xs_v[e, sl]
      return carry2

    lax.fori_loop(0, BD // 16, group_body, 0)
    pltpu.async_copy(xs_v, accum.at[pk_v.at[1]], semS, add=True)

  def drain_a(buf):
    pk_v, gidx_v, hs_v, xs_v, sem, semS = buf
    pltpu.make_async_copy(xs_v, accum.at[pk_v.at[1]], semS).wait()

  def fetch_b(vb, buf):
    pk_v, gidx_v, hs_v, xs_v, sem, semS = buf

    @pl.when(vb >= 2)
    def _drain_scatter():
      pltpu.make_async_copy(hs_v, accum.at[pk_v.at[0]], semS).wait()

    wj, b = wjb(vb)
    pltpu.sync_copy(pk2.at[wj, b], pk_v)
    for k in range(BD // 16):
      sl = pl.ds(k * 16, 16)
      gidx_v[sl] = pk_v[1, sl] + row_off
    pltpu.async_copy(xcat.at[gidx_v], hs_v, sem)

  def process_b(buf):
    pk_v, gidx_v, hs_v, xs_v, sem, semS = buf
    pltpu.make_async_copy(xcat.at[gidx_v], hs_v, sem).wait()

    def group_body(g, carry2):
      b16 = plsc.bitcast(pk_v[3, pl.ds(g * 16, 16)], jnp.float32)
      for e16 in range(16):
        e = g * 16 + e16
        be = -b16[e16]
        for k in range(DH // 16):
          sl = pl.ds(k * 16, 16)
          hs_v[e, sl] = be * hs_v[e, sl]
      return carry2

    lax.fori_loop(0, BD // 16, group_body, 0)
    pltpu.async_copy(hs_v, accum.at[pk_v.at[0]], semS, add=True)

  def drain_b(buf):
    pk_v, gidx_v, hs_v, xs_v, sem, semS = buf
    pltpu.make_async_copy(hs_v, accum.at[pk_v.at[0]], semS).wait()

  NBT = 2 * NBD
  NG = NBT // 2

  def run_pass(fetch, process, drain):
    fetch(0, bufs[0])

    def pair_body(g, carry):
      fetch(2 * g + 1, bufs[1])
      process(bufs[0])

      @pl.when(g < NG - 1)
      def _prefetch():
        fetch(2 * g + 2, bufs[0])

      process(bufs[1])
      return carry

    lax.fori_loop(0, NG, pair_body, 0)
    drain(bufs[0])
    drain(bufs[1])

  run_pass(fetch_a, process_a, drain_a)
  run_pass(fetch_b, process_b, drain_b)

  plsc.subcore_barrier()

  rsl = pl.ds(s * ROWS_ACC, ROWS_ACC)
  pltpu.sync_copy(accum.at[rsl], acc_out.at[c, rsl])


def _p2(xcat, hcat, pk2, zeros_nd):
  f32 = jnp.float32
  i32 = jnp.int32
  dbuf = [
      pltpu.VMEM((4, BD), i32),
      pltpu.VMEM((BD,), i32),
      pltpu.VMEM((BD, DH), f32),
      pltpu.VMEM((BD, DH), f32),
  ]
  return pl.kernel(
      _p2_body,
      out_type=jax.ShapeDtypeStruct((NC, N_ACC, DH), f32),
      mesh=_mesh,
      compiler_params=_sc_params,
      scratch_types=dbuf + dbuf + [
          pltpu.VMEM_SHARED((N_ACC, DH), f32),
          pltpu.SemaphoreType.DMA,
          pltpu.SemaphoreType.DMA,
          pltpu.SemaphoreType.DMA,
          pltpu.SemaphoreType.DMA,
      ],
  )(xcat, hcat, pk2, zeros_nd)


def _combine_body(acc_ref, h_ref, x_ref, dsq_ref, s_ref, b_ref, o_ref):
  dsq = dsq_ref[...]
  sc = s_ref[0] + s_ref[1]
  bias = b_ref[...]
  o_ref[:, :DH] = (acc_ref[0] + dsq * h_ref[0] + sc * x_ref[:, :DH]
                   + bias[:, :DH])
  o_ref[:, DH:] = (acc_ref[1] + dsq * h_ref[1] + sc * x_ref[:, DH:]
                   + bias[:, DH:])


def _combine(acc, h3, x, dsq_col, s_part, b2d):
  RT = 400
  f32 = jnp.float32
  return pl.pallas_call(
      _combine_body,
      grid=(N // RT,),
      in_specs=[
          pl.BlockSpec((NC, RT, DH), lambda i: (0, i, 0)),
          pl.BlockSpec((NC, RT, DH), lambda i: (0, i, 0)),
          pl.BlockSpec((RT, D), lambda i: (i, 0)),
          pl.BlockSpec((RT, 1), lambda i: (i, 0)),
          pl.BlockSpec((NC, RT, 1), lambda i: (0, i, 0)),
          pl.BlockSpec((1, D), lambda i: (0, 0)),
      ],
      out_specs=pl.BlockSpec((RT, D), lambda i: (i, 0)),
      out_shape=jax.ShapeDtypeStruct((N, D), f32),
  )(acc, h3, x, dsq_col, s_part, b2d)


@jax.jit
def _run(x, edge_index, A, weight, temperature, norm_energies, W, b):
  f32 = jnp.float32
  i32 = jnp.int32
  src = edge_index[0]
  dst = edge_index[1]

  pad = E_PAD - E
  shp128 = (NW, NBLK, BLOCK)
  shp64 = (NW, NBD, BD)
  src_f = jnp.concatenate([src, jnp.zeros((pad,), i32)])
  dst_f = jnp.concatenate([dst, jnp.zeros((pad,), i32)])
  a_f = jnp.concatenate([A, jnp.zeros((pad,), f32)])
  flag_f = jnp.concatenate([jnp.ones((E,), f32), jnp.zeros((pad,), f32)])
  zeros_np = jnp.zeros((N_PAD,), f32)
  zeros_nd = jnp.zeros((N_ACC, DH), f32)

  h3 = _matmul(x, W)
  hcat = h3.reshape(2 * N, DH)
  xcat = x.reshape(N, 2, DH).transpose(1, 0, 2).reshape(2 * N, DH)

  shp1 = (NW, NB1, BP1)
  pk1 = jnp.stack(
      [src_f.reshape(shp1), dst_f.reshape(shp1),
       lax.bitcast_convert_type(a_f, i32).reshape(shp1),
       lax.bitcast_convert_type(flag_f, i32).reshape(shp1)], axis=2)

  xbi = lax.bitcast_convert_type(
      x.astype(jnp.bfloat16).reshape(N, D // 2, 2), i32).reshape(N, D // 2)
  en_part, deg_part = _p1(xbi, pk1, zeros_np)
  dinv2d, cn2d, dsq2d = _node_math(en_part, deg_part, temperature, weight,
                                   norm_energies)
  dinv_n = dinv2d.reshape(-1)[:N]
  cn_n = cn2d.reshape(-1)[:N]

  alphap, svalp, s_part = _p15(src_f.reshape(shp128), dst_f.reshape(shp128),
                               a_f.reshape(shp128), flag_f.reshape(shp128),
                               dinv_n, cn_n, zeros_np)
  pk2 = jnp.stack(
      [src_f.reshape(shp64), dst_f.reshape(shp64),
       lax.bitcast_convert_type(alphap, i32).reshape(shp64),
       lax.bitcast_convert_type(svalp, i32).reshape(shp64)], axis=2)
  acc = _p2(xcat, hcat, pk2, zeros_nd)

  out = _combine(acc, h3, x, dsq2d.reshape(-1)[:N].reshape(N, 1),
                 s_part[:, :N].reshape(NC, N, 1), b.reshape(1, D))
  return out


def kernel(x, edge_index, A, weight, temperature, norm_energies, W, b):
  return _run(x, edge_index, A, weight, temperature, norm_energies, W, b)
